# baseline (speedup 1.0000x reference)
import functools

import jax
import jax.numpy as jnp
from jax import lax
from jax.experimental import pallas as pl
from jax.experimental.pallas import tpu as pltpu
from jax.experimental.pallas import tpu_sc as plsc

N = 10000
E = 320000
H = 128
C5 = 10000
C6 = 10000
A = 5 * C5 + 6 * C6

_BE = 2000

_P5 = 57600
_AP = _P5 + 60032
_TA5 = _P5 // 8
_TA6 = (_AP - _P5) // 8
_LROWS = _AP + 16

_NC, _NS = 2, 16
_NW = _NC * _NS
_SC_MESH = plsc.VectorSubcoreMesh(core_axis_name="c", subcore_axis_name="s")


def _ne_lift_sc(node_rep, edge_src, edge_dst):
    epw = E // _NW
    gb = 200
    nb = epw // gb
    npr = 1000

    @functools.partial(
        pl.kernel,
        out_type=jax.ShapeDtypeStruct((E, H), jnp.float32),
        mesh=_SC_MESH,
        scratch_types=[
            pltpu.VMEM((gb,), jnp.int32),
            pltpu.VMEM((gb,), jnp.int32),
            pltpu.VMEM((gb, H), jnp.float32),
            pltpu.VMEM_SHARED((N, H), jnp.float32),
            pltpu.SemaphoreType.DMA,
        ],
    )
    def k(node_hbm, src_hbm, dst_hbm, out_hbm, idx_s, idx_d, rows, nodes_sh,
          sem):
        c = lax.axis_index("c")
        s = lax.axis_index("s")
        wid = c * _NS + s

        @pl.when(s < N // npr)
        def _():
            pltpu.sync_copy(node_hbm.at[pl.ds(s * npr, npr)],
                            nodes_sh.at[pl.ds(s * npr, npr)])

        plsc.subcore_barrier()
        base = wid * epw

        def body(b, carry):
            off = base + b * gb
            pltpu.sync_copy(src_hbm.at[pl.ds(off, gb)], idx_s)
            pltpu.sync_copy(dst_hbm.at[pl.ds(off, gb)], idx_d)
            pltpu.async_copy(nodes_sh.at[idx_s], rows, sem).wait()
            pltpu.async_copy(nodes_sh.at[idx_d], rows, sem, add=True).wait()
            pltpu.sync_copy(rows, out_hbm.at[pl.ds(off, gb)])
            return carry

        lax.fori_loop(0, nb, body, 0)

    return k(node_rep, edge_src, edge_dst)


def _node_aggr_sc(e_mid, edge_src, edge_dst):
    epw = E // _NW
    gb = 200
    nb = epw // gb
    npr = 1000
    zr = 40

    @functools.partial(
        pl.kernel,
        out_type=jax.ShapeDtypeStruct((_NC, N, H), jnp.float32),
        mesh=_SC_MESH,
        scratch_types=[
            pltpu.VMEM((gb,), jnp.int32),
            pltpu.VMEM((gb,), jnp.int32),
            pltpu.VMEM((gb, H), jnp.float32),
            pltpu.VMEM((zr, H), jnp.float32),
            pltpu.VMEM_SHARED((N, H), jnp.float32),
            pltpu.SemaphoreType.DMA,
        ],
    )
    def k(emid_hbm, src_hbm, dst_hbm, out_hbm, idx_s, idx_d, rows, zbuf, acc,
          sem):
        c = lax.axis_index("c")
        s = lax.axis_index("s")
        wid = c * _NS + s

        def zv(t, carry):
            i = t // 8
            j = t - i * 8
            zbuf[i, pl.ds(j * 16, 16)] = jnp.zeros((16,), jnp.float32)
            return carry

        lax.fori_loop(0, zr * 8, zv, 0)

        @pl.when(s < N // npr)
        def _():
            def zc(r, carry):
                pltpu.sync_copy(zbuf, acc.at[pl.ds(s * npr + r * zr, zr)])
                return carry

            lax.fori_loop(0, npr // zr, zc, 0)

        plsc.subcore_barrier()
        base = wid * epw

        def body(b, carry):
            off = base + b * gb
            pltpu.sync_copy(src_hbm.at[pl.ds(off, gb)], idx_s)
            pltpu.sync_copy(dst_hbm.at[pl.ds(off, gb)], idx_d)
            pltpu.sync_copy(emid_hbm.at[pl.ds(off, gb)], rows)
            pltpu.sync_copy(rows, acc.at[idx_s], add=True)
            pltpu.sync_copy(rows, acc.at[idx_d], add=True)
            return carry

        lax.fori_loop(0, nb, body, 0)
        plsc.subcore_barrier()

        @pl.when(s < N // npr)
        def _():
            pltpu.sync_copy(acc.at[pl.ds(s * npr, npr)],
                            out_hbm.at[c, pl.ds(s * npr, npr)])

    return k(e_mid, edge_src, edge_dst)


def _e2c_sc(edge_rep, a5, a6):
    gb5, gb6 = 400, 480
    nblk5 = (5 * C5) // gb5
    nblk6 = (6 * C6) // gb6

    @functools.partial(
        pl.kernel,
        out_type=[
            jax.ShapeDtypeStruct((5 * C5, H), jnp.float32),
            jax.ShapeDtypeStruct((6 * C6, H), jnp.float32),
        ],
        mesh=_SC_MESH,
        scratch_types=[
            pltpu.VMEM((gb5,), jnp.int32),
            pltpu.VMEM((gb6,), jnp.int32),
            pltpu.VMEM((gb6, H), jnp.float32),
            pltpu.SemaphoreType.DMA,
        ],
    )
    def k(er_hbm, a5_hbm, a6_hbm, o5_hbm, o6_hbm, i5, i6, rows, sem):
        c = lax.axis_index("c")
        s = lax.axis_index("s")
        wid = c * _NS + s

        @pl.when(wid < 16)
        def _():
            def body5(t, carry):
                off = (wid + t * 16) * gb5
                pltpu.sync_copy(a5_hbm.at[pl.ds(off, gb5)], i5)
                pltpu.async_copy(er_hbm.at[i5], rows.at[pl.ds(0, gb5)],
                                 sem).wait()
                pltpu.sync_copy(rows.at[pl.ds(0, gb5)],
                                o5_hbm.at[pl.ds(off, gb5)])
                return carry

            lax.fori_loop(0, (nblk5 - wid + 15) // 16, body5, 0)

        @pl.when(wid >= 16)
        def _():
            def body6(t, carry):
                off = (wid - 16 + t * 16) * gb6
                pltpu.sync_copy(a6_hbm.at[pl.ds(off, gb6)], i6)
                pltpu.async_copy(er_hbm.at[i6], rows, sem).wait()
                pltpu.sync_copy(rows, o6_hbm.at[pl.ds(off, gb6)])
                return carry

            lax.fori_loop(0, (nblk6 - (wid - 16) + 15) // 16, body6, 0)

    return k(edge_rep, a5, a6)


def _dot(x, w):
    return jnp.dot(x.astype(jnp.bfloat16), w.astype(jnp.bfloat16),
                   preferred_element_type=jnp.float32)


def _iota16():
    return lax.iota(jnp.int32, 16)


def _zero_fill(zbuf, zr):
    def zv(t, carry):
        i = t // 8
        j = t - i * 8
        zbuf[i, pl.ds(j * 16, 16)] = jnp.zeros((16,), jnp.float32)
        return carry

    lax.fori_loop(0, zr * 8, zv, 0)


def _stage_atoms(ape_hbm, aidx, s):
    @pl.when(s < 8)
    def _():
        pltpu.sync_copy(ape_hbm.at[pl.ds(s * _TA5, _TA5)],
                        aidx.at[pl.ds(0, _TA5)])

    @pl.when(s >= 8)
    def _():
        pltpu.sync_copy(ape_hbm.at[pl.ds(_P5 + (s - 8) * _TA6, _TA6)],
                        aidx.at[pl.ds(0, _TA6)])


_LANE15 = None


def _splat_last(pc):
    return jnp.take(pc, jnp.full((16,), 15, jnp.int32))


def _compress(aidx, clist, plist, la, ta, tbase, srcoff, lo, hi, it16,
              double_out=False):
    junk = la - 16

    def cbody(j, cnt_v):
        v = aidx[pl.ds(j * 16, 16)]
        m = (v >= lo) & (v < hi)
        pc = plsc.cumsum(m.astype(jnp.int32))
        dest = jnp.where(m, cnt_v + pc - 1, junk + it16)
        cval = v - lo
        pos = tbase + j * 16 + it16 - srcoff
        if double_out:
            cval = cval + cval
            pos = pos + pos
        plsc.store_scatter(clist, [dest], cval)
        plsc.store_scatter(plist, [dest], pos)
        return cnt_v + _splat_last(pc)

    return lax.fori_loop(0, ta // 16, cbody, jnp.zeros((16,), jnp.int32))


def _refresh(dst, lst, b, gb, cnt_v, padbase, shift, it16):
    def cp(jj, carry):
        g = b * gb + jj * 16
        v = lst[pl.ds(g, 16)]
        pos16 = g + it16
        dst[pl.ds(jj * 16, 16)] = jnp.where(pos16 < cnt_v, v + shift,
                                            padbase + it16)
        return carry

    lax.fori_loop(0, gb // 16, cp, 0)


_CH1 = 10240
_GB1 = 128


def _edge_scatter_sc(lvl_all, ape, zeros):
    la = _TA6 + 2 * _GB1

    @functools.partial(
        pl.kernel,
        out_type=jax.ShapeDtypeStruct((E, H), jnp.float32),
        mesh=_SC_MESH,
        compiler_params=pltpu.CompilerParams(needs_layout_passes=False),
        scratch_types=[
            pltpu.VMEM((_TA6,), jnp.int32),
            pltpu.VMEM((la,), jnp.int32),
            pltpu.VMEM((la,), jnp.int32),
            pltpu.VMEM((_GB1,), jnp.int32),
            pltpu.VMEM((_GB1,), jnp.int32),
            pltpu.VMEM((_GB1, H), jnp.float32),
            pltpu.VMEM((40, H), jnp.float32),
            pltpu.VMEM_SHARED((_CH1 + 16, H), jnp.float32),
            pltpu.SemaphoreType.DMA,
        ],
    )
    def k(lvl_hbm, ape_hbm, z_hbm, out_hbm, aidx, clist, plist, cbuf, pbuf,
          rows, zbuf, acc, sem):
        c = lax.axis_index("c")
        s = lax.axis_index("s")
        it16 = _iota16()
        pltpu.sync_copy(z_hbm, zbuf)
        _stage_atoms(ape_hbm, aidx, s)
        ta = jnp.where(s < 8, _TA5, _TA6)
        tbase = jnp.where(s < 8, s * _TA5, _P5 + (s - 8) * _TA6)

        def one_pass(p, carry):
            lo = (c * 16 + p) * _CH1
            hi = lo + _CH1
            plsc.subcore_barrier()
            zd = [
                pltpu.async_copy(
                    zbuf, acc.at[pl.ds(s * 640 + r * 40, 40)], sem)
                for r in range(16)
            ]
            cnt_v = _compress(aidx, clist, plist, la, ta, tbase, 0,
                              lo, hi, it16)
            for d in zd:
                d.wait()
            plsc.subcore_barrier()

            def sbc(b):
                return jnp.any(cnt_v > b * _GB1)

            def sb(b):
                _refresh(cbuf, clist, b, _GB1, cnt_v, _CH1, 0, it16)
                _refresh(pbuf, plist, b, _GB1, cnt_v, 0, 0, it16)
                pltpu.async_copy(lvl_hbm.at[pbuf], rows, sem).wait()
                pltpu.sync_copy(rows, acc.at[cbuf], add=True)
                return b + 1

            lax.while_loop(sbc, sb, jnp.int32(0))
            plsc.subcore_barrier()

            @pl.when(lo + s * 640 < E)
            def _():
                pltpu.sync_copy(acc.at[pl.ds(s * 640, 640)],
                                out_hbm.at[pl.ds(lo + s * 640, 640)])

            return carry

        lax.fori_loop(0, 16, one_pass, 0)

    return k(lvl_all, ape, zeros)


_CH2 = 4800
_GB2 = 128


def _linmap_sc(cyc2, ape, zeros):
    la = _TA6 + 2 * _GB2

    @functools.partial(
        pl.kernel,
        out_type=jax.ShapeDtypeStruct((2 * _LROWS, H), jnp.float32),
        mesh=_SC_MESH,
        compiler_params=pltpu.CompilerParams(needs_layout_passes=False),
        scratch_types=[
            pltpu.VMEM((_TA6,), jnp.int32),
            pltpu.VMEM((la,), jnp.int32),
            pltpu.VMEM((la,), jnp.int32),
            pltpu.VMEM((_GB2,), jnp.int32),
            pltpu.VMEM((_GB2,), jnp.int32),
            pltpu.VMEM((_GB2, H), jnp.float32),
            pltpu.VMEM((40, H), jnp.float32),
            pltpu.VMEM_SHARED((2 * _CH2 + 16, H), jnp.float32),
            pltpu.SemaphoreType.DMA,
        ],
    )
    def k(cyc_hbm, ape_hbm, z_hbm, lin_hbm, aidx, clist, plist, cbuf, pbuf,
          rows, zbuf, acc, sem):
        c = lax.axis_index("c")
        s = lax.axis_index("s")
        it16 = _iota16()
        pltpu.sync_copy(z_hbm, zbuf)
        _stage_atoms(ape_hbm, aidx, s)
        ta = jnp.where(s < 8, _TA5, _TA6)
        tbase = jnp.where(s < 8, s * _TA5, _P5 + (s - 8) * _TA6)
        srccor = jnp.where(s < 8, 0, _P5 - 5 * C5)

        def one_pass(p, carry):
            lo = (c * 34 + p) * _CH2
            hi = lo + _CH2
            plsc.subcore_barrier()
            zd = [
                pltpu.async_copy(
                    zbuf, acc.at[pl.ds(s * 600 + r * 40, 40)], sem)
                for r in range(15)
            ]
            cnt_v = _compress(aidx, clist, plist, la, ta, tbase, srccor,
                              lo, hi, it16, double_out=True)
            for d in zd:
                d.wait()
            plsc.subcore_barrier()

            def sbc(b):
                return jnp.any(cnt_v > b * _GB2)

            def sb(b):
                for half in (0, 1):
                    _refresh(cbuf, clist, b, _GB2, cnt_v, 2 * _CH2 + half,
                             half, it16)
                    _refresh(pbuf, plist, b, _GB2, cnt_v, half, half, it16)
                    pltpu.async_copy(cyc_hbm.at[pbuf], rows, sem).wait()
                    pltpu.sync_copy(rows, acc.at[cbuf], add=True)
                return b + 1

            lax.while_loop(sbc, sb, jnp.int32(0))
            plsc.subcore_barrier()

            def sb2(b):
                for half in (0, 1):
                    _refresh(cbuf, clist, b, _GB2, cnt_v, 2 * _CH2 + half,
                             half, it16)
                    _refresh(pbuf, plist, b, _GB2, cnt_v, 2 * _AP + half,
                             srccor + srccor + half, it16)
                    pltpu.sync_copy(acc.at[cbuf], rows)
                    pltpu.sync_copy(rows, lin_hbm.at[pbuf])
                return b + 1

            lax.while_loop(sbc, sb2, jnp.int32(0))
            return carry

        lax.fori_loop(0, 34 - c, one_pass, 0)

    return k(cyc2, ape, zeros)


def _ne_body(lift_ref, er_ref, w1_ref, wl_ref, eps2_ref, emid_ref, eo1_ref):
    lift = lift_ref[...]
    w1 = w1_ref[...]
    e_mid = jnp.maximum(
        _dot(lift, w1[:H])
        + _dot(er_ref[...], w1[H:]),
        0.0,
    )
    emid_ref[...] = e_mid
    eo1_ref[...] = jnp.maximum(
        _dot((1.0 + eps2_ref[0, 0]) * e_mid + lift, wl_ref[...]),
        0.0,
    )


def _stage_ne(ne_lift, edge_rep, W_ne_lvl1, W_ne_lift, eps_ne_2):
    grid = E // _BE
    return pl.pallas_call(
        _ne_body,
        grid=(grid,),
        in_specs=[
            pl.BlockSpec((_BE, H), lambda i: (i, 0)),
            pl.BlockSpec((_BE, H), lambda i: (i, 0)),
            pl.BlockSpec((2 * H, H), lambda i: (0, 0)),
            pl.BlockSpec((H, H), lambda i: (0, 0)),
            pl.BlockSpec((1, 1), lambda i: (0, 0), memory_space=pltpu.SMEM),
        ],
        out_specs=[
            pl.BlockSpec((_BE, H), lambda i: (i, 0)),
            pl.BlockSpec((_BE, H), lambda i: (i, 0)),
        ],
        out_shape=[
            jax.ShapeDtypeStruct((E, H), jnp.float32),
            jax.ShapeDtypeStruct((E, H), jnp.float32),
        ],
    )(ne_lift, edge_rep, W_ne_lvl1, W_ne_lift, eps_ne_2.reshape(1, 1))


def _node_body(nr_ref, p0_ref, p1_ref, w_ref, eps_ref, out_ref):
    x = ((1.0 + eps_ref[0, 0]) * nr_ref[...] + p0_ref[0] + p1_ref[0])
    out_ref[...] = jnp.maximum(
        _dot(x, w_ref[...]), 0.0)


def _stage_node(node_rep, partials, W_ne_lvl2, eps_ne_1):
    bn = 2000
    return pl.pallas_call(
        _node_body,
        grid=(N // bn,),
        in_specs=[
            pl.BlockSpec((bn, H), lambda i: (i, 0)),
            pl.BlockSpec((1, bn, H), lambda i: (0, i, 0)),
            pl.BlockSpec((1, bn, H), lambda i: (1, i, 0)),
            pl.BlockSpec((H, H), lambda i: (0, 0)),
            pl.BlockSpec((1, 1), lambda i: (0, 0), memory_space=pltpu.SMEM),
        ],
        out_specs=pl.BlockSpec((bn, H), lambda i: (i, 0)),
        out_shape=jax.ShapeDtypeStruct((N, H), jnp.float32),
    )(node_rep, partials, partials, W_ne_lvl2, eps_ne_1.reshape(1, 1))


def _lvl1_body(k, e2c_ref, cyc_ref, w_ref, out_ref):
    bc = e2c_ref.shape[0] // k
    e2c = e2c_ref[...]
    s = jnp.sum(e2c.reshape(bc, k, H), axis=1)
    bsum = jnp.broadcast_to(s[:, None, :], (bc, k, H)).reshape(bc * k, H)
    w = w_ref[...]
    out_ref[...] = jnp.maximum(
        _dot(e2c, w[:H])
        + _dot(bsum, w[H:2 * H])
        + _dot(cyc_ref[...], w[2 * H:]),
        0.0,
    )


def _stage_lvl1(k, nc, e2c, cyc, W_ec_lvl1, off_blocks, buf=None):
    bc = 200
    rows = bc * k
    body = functools.partial(_lvl1_body, k)
    in_specs = [
        pl.BlockSpec((rows, H), lambda i: (i, 0)),
        pl.BlockSpec((rows, 2 * H), lambda i: (i, 0)),
        pl.BlockSpec((4 * H, H), lambda i: (0, 0)),
    ]
    args = [e2c, cyc, W_ec_lvl1]
    kwargs = {}
    if buf is not None:
        in_specs.append(pl.BlockSpec(memory_space=pl.MemorySpace.ANY))
        args.append(buf)
        kwargs["input_output_aliases"] = {3: 0}
        body = functools.partial(_lvl1_body_alias, k)
    return pl.pallas_call(
        body,
        grid=(nc // bc,),
        in_specs=in_specs,
        out_specs=pl.BlockSpec((rows, H),
                               lambda i: (off_blocks + i, 0)),
        out_shape=jax.ShapeDtypeStruct((_AP, H), jnp.float32),
        **kwargs,
    )(*args)


def _lvl1_body_alias(k, e2c_ref, cyc_ref, w_ref, buf_ref, out_ref):
    _lvl1_body(k, e2c_ref, cyc_ref, w_ref, out_ref)


def _eo_body(er_ref, lae_ref, eo1_ref, w2_ref, wm_ref, e11_ref, e12_ref,
             out_ref):
    eo2 = jnp.maximum(
        _dot((1.0 + e11_ref[0, 0]) * er_ref[...]
                + (1.0 + e12_ref[0, 0]) * lae_ref[...], w2_ref[...]),
        0.0,
    )
    wm = wm_ref[...]
    out_ref[...] = jnp.maximum(
        _dot(eo1_ref[...], wm[:H])
        + _dot(eo2, wm[H:]),
        0.0,
    )


def _stage_edge_out(edge_rep, lvl_aggr_e, edge_out_1, W_ec_lvl2, W_mlp,
                    eps_ec_11, eps_ec_12):
    return pl.pallas_call(
        _eo_body,
        grid=(E // _BE,),
        in_specs=[
            pl.BlockSpec((_BE, H), lambda i: (i, 0)),
            pl.BlockSpec((_BE, H), lambda i: (i, 0)),
            pl.BlockSpec((_BE, H), lambda i: (i, 0)),
            pl.BlockSpec((H, H), lambda i: (0, 0)),
            pl.BlockSpec((2 * H, H), lambda i: (0, 0)),
            pl.BlockSpec((1, 1), lambda i: (0, 0), memory_space=pltpu.SMEM),
            pl.BlockSpec((1, 1), lambda i: (0, 0), memory_space=pltpu.SMEM),
        ],
        out_specs=pl.BlockSpec((_BE, H), lambda i: (i, 0)),
        out_shape=jax.ShapeDtypeStruct((E, H), jnp.float32),
    )(edge_rep, lvl_aggr_e, edge_out_1, W_ec_lvl2, W_mlp,
      eps_ec_11.reshape(1, 1), eps_ec_12.reshape(1, 1))


def _cyc_body(k, lin_ref, e2c_ref, w_ref, eps_ref, out_ref):
    bc = e2c_ref.shape[0] // k
    e2c = e2c_ref[...]
    s = jnp.sum(e2c.reshape(bc, k, H), axis=1)
    bsum = jnp.broadcast_to(s[:, None, :], (bc, k, H)).reshape(bc * k, H)
    w = w_ref[...]
    out_ref[...] = jnp.maximum(
        (1.0 + eps_ref[0, 0])
        * _dot(lin_ref[...], w)
        + _dot(e2c, w[:H])
        + _dot(bsum, w[H:]),
        0.0,
    )


def _stage_cycle_out(k, nc, linmap, off_blocks, e2c, W_ec_lift, eps_ec_2):
    bc = 200
    rows = bc * k
    return pl.pallas_call(
        functools.partial(_cyc_body, k),
        grid=(nc // bc,),
        in_specs=[
            pl.BlockSpec((rows, 2 * H), lambda i: (off_blocks + i, 0)),
            pl.BlockSpec((rows, H), lambda i: (i, 0)),
            pl.BlockSpec((2 * H, 2 * H), lambda i: (0, 0)),
            pl.BlockSpec((1, 1), lambda i: (0, 0), memory_space=pltpu.SMEM),
        ],
        out_specs=pl.BlockSpec((rows, 2 * H), lambda i: (i, 0)),
        out_shape=jax.ShapeDtypeStruct((nc * k, 2 * H), jnp.float32),
    )(linmap, e2c, W_ec_lift, eps_ec_2.reshape(1, 1))


def kernel(node_rep, edge_rep, cycle_rep, edge_src, edge_dst, cycle5_edges,
           cycle6_edges, W_ne_lift, W_ne_lvl1, W_ne_lvl2, W_ec_lift,
           W_ec_lvl1, W_ec_lvl2, W_mlp, eps_ne_1, eps_ne_2, eps_ec_11,
           eps_ec_12, eps_ec_2):
    ne_lift = _ne_lift_sc(node_rep, edge_src, edge_dst)
    e_mid, edge_out_1 = _stage_ne(ne_lift, edge_rep, W_ne_lvl1, W_ne_lift,
                                  eps_ne_2)
    partials = _node_aggr_sc(e_mid, edge_src, edge_dst)
    node_out = _stage_node(node_rep, partials, W_ne_lvl2, eps_ne_1)

    a5 = cycle5_edges.reshape(-1)
    a6 = cycle6_edges.reshape(-1)
    e2c5, e2c6 = _e2c_sc(edge_rep, a5, a6)
    cyc5 = cycle_rep[:5 * C5]
    cyc6 = cycle_rep[5 * C5:]
    lvl_c5 = _stage_lvl1(5, C5, e2c5, cyc5, W_ec_lvl1, 0)
    lvl_all = _stage_lvl1(6, C6, e2c6, cyc6, W_ec_lvl1, _P5 // 1200,
                          buf=lvl_c5)

    sentinel = jnp.int32(1 << 20)
    ape = jnp.concatenate([
        a5, jnp.full((_P5 - 5 * C5,), sentinel, jnp.int32),
        a6, jnp.full((_AP - _P5 - 6 * C6,), sentinel, jnp.int32)])
    zeros40 = jnp.zeros((40, H), jnp.float32)
    lvl_aggr_e = _edge_scatter_sc(lvl_all, ape, zeros40)
    lin2 = _linmap_sc(cycle_rep.reshape(2 * A, H), ape, zeros40)
    linmap = lin2.reshape(_LROWS, 2 * H)

    edge_out = _stage_edge_out(edge_rep, lvl_aggr_e, edge_out_1, W_ec_lvl2,
                               W_mlp, eps_ec_11, eps_ec_12)
    co5 = _stage_cycle_out(5, C5, linmap, 0, e2c5, W_ec_lift, eps_ec_2)
    co6 = _stage_cycle_out(6, C6, linmap, _P5 // 1200, e2c6, W_ec_lift,
                           eps_ec_2)
    cycle_out = jnp.concatenate([co5, co6], axis=0)
    return (node_out, edge_out, cycle_out)


# speedup vs baseline: 1.0165x; 1.0165x over previous
import functools

import jax
import jax.numpy as jnp
from jax import lax
from jax.experimental import pallas as pl
from jax.experimental.pallas import tpu as pltpu
from jax.experimental.pallas import tpu_sc as plsc

N = 10000
E = 320000
H = 128
C5 = 10000
C6 = 10000
A = 5 * C5 + 6 * C6

_BE = 2000

_P5 = 57600
_AP = _P5 + 60032
_TA5 = _P5 // 8
_TA6 = (_AP - _P5) // 8
_LROWS = _AP + 16

_NC, _NS = 2, 16
_NW = _NC * _NS
_SC_MESH = plsc.VectorSubcoreMesh(core_axis_name="c", subcore_axis_name="s")


def _ne_lift_sc(node_rep, edge_src, edge_dst):
    epw = E // _NW
    gb = 200
    nb = epw // gb
    npr = 1000

    @functools.partial(
        pl.kernel,
        out_type=jax.ShapeDtypeStruct((E, H), jnp.float32),
        mesh=_SC_MESH,
        scratch_types=[
            pltpu.VMEM((gb,), jnp.int32),
            pltpu.VMEM((gb,), jnp.int32),
            pltpu.VMEM((gb, H), jnp.float32),
            pltpu.VMEM_SHARED((N, H), jnp.float32),
            pltpu.SemaphoreType.DMA,
        ],
    )
    def k(node_hbm, src_hbm, dst_hbm, out_hbm, idx_s, idx_d, rows, nodes_sh,
          sem):
        c = lax.axis_index("c")
        s = lax.axis_index("s")
        wid = c * _NS + s

        @pl.when(s < N // npr)
        def _():
            pltpu.sync_copy(node_hbm.at[pl.ds(s * npr, npr)],
                            nodes_sh.at[pl.ds(s * npr, npr)])

        plsc.subcore_barrier()
        base = wid * epw

        def body(b, carry):
            off = base + b * gb
            pltpu.sync_copy(src_hbm.at[pl.ds(off, gb)], idx_s)
            pltpu.sync_copy(dst_hbm.at[pl.ds(off, gb)], idx_d)
            pltpu.async_copy(nodes_sh.at[idx_s], rows, sem).wait()
            pltpu.async_copy(nodes_sh.at[idx_d], rows, sem, add=True).wait()
            pltpu.sync_copy(rows, out_hbm.at[pl.ds(off, gb)])
            return carry

        lax.fori_loop(0, nb, body, 0)

    return k(node_rep, edge_src, edge_dst)


def _node_aggr_sc(e_mid, edge_src, edge_dst):
    epw = E // _NW
    gb = 200
    nb = epw // gb
    npr = 1000
    zr = 40

    @functools.partial(
        pl.kernel,
        out_type=jax.ShapeDtypeStruct((_NC, N, H), jnp.float32),
        mesh=_SC_MESH,
        scratch_types=[
            pltpu.VMEM((gb,), jnp.int32),
            pltpu.VMEM((gb,), jnp.int32),
            pltpu.VMEM((gb, H), jnp.float32),
            pltpu.VMEM((zr, H), jnp.float32),
            pltpu.VMEM_SHARED((N, H), jnp.float32),
            pltpu.SemaphoreType.DMA,
        ],
    )
    def k(emid_hbm, src_hbm, dst_hbm, out_hbm, idx_s, idx_d, rows, zbuf, acc,
          sem):
        c = lax.axis_index("c")
        s = lax.axis_index("s")
        wid = c * _NS + s

        def zv(t, carry):
            i = t // 8
            j = t - i * 8
            zbuf[i, pl.ds(j * 16, 16)] = jnp.zeros((16,), jnp.float32)
            return carry

        lax.fori_loop(0, zr * 8, zv, 0)

        @pl.when(s < N // npr)
        def _():
            def zc(r, carry):
                pltpu.sync_copy(zbuf, acc.at[pl.ds(s * npr + r * zr, zr)])
                return carry

            lax.fori_loop(0, npr // zr, zc, 0)

        plsc.subcore_barrier()
        base = wid * epw

        def body(b, carry):
            off = base + b * gb
            pltpu.sync_copy(src_hbm.at[pl.ds(off, gb)], idx_s)
            pltpu.sync_copy(dst_hbm.at[pl.ds(off, gb)], idx_d)
            pltpu.sync_copy(emid_hbm.at[pl.ds(off, gb)], rows)
            pltpu.sync_copy(rows, acc.at[idx_s], add=True)
            pltpu.sync_copy(rows, acc.at[idx_d], add=True)
            return carry

        lax.fori_loop(0, nb, body, 0)
        plsc.subcore_barrier()

        @pl.when(s < N // npr)
        def _():
            pltpu.sync_copy(acc.at[pl.ds(s * npr, npr)],
                            out_hbm.at[c, pl.ds(s * npr, npr)])

    return k(e_mid, edge_src, edge_dst)


def _e2c_sc(edge_rep, a5, a6):
    gb5, gb6 = 400, 480
    nblk5 = (5 * C5) // gb5
    nblk6 = (6 * C6) // gb6

    @functools.partial(
        pl.kernel,
        out_type=[
            jax.ShapeDtypeStruct((5 * C5, H), jnp.float32),
            jax.ShapeDtypeStruct((6 * C6, H), jnp.float32),
        ],
        mesh=_SC_MESH,
        scratch_types=[
            pltpu.VMEM((gb5,), jnp.int32),
            pltpu.VMEM((gb6,), jnp.int32),
            pltpu.VMEM((gb6, H), jnp.float32),
            pltpu.SemaphoreType.DMA,
        ],
    )
    def k(er_hbm, a5_hbm, a6_hbm, o5_hbm, o6_hbm, i5, i6, rows, sem):
        c = lax.axis_index("c")
        s = lax.axis_index("s")
        wid = c * _NS + s

        @pl.when(wid < 16)
        def _():
            def body5(t, carry):
                off = (wid + t * 16) * gb5
                pltpu.sync_copy(a5_hbm.at[pl.ds(off, gb5)], i5)
                pltpu.async_copy(er_hbm.at[i5], rows.at[pl.ds(0, gb5)],
                                 sem).wait()
                pltpu.sync_copy(rows.at[pl.ds(0, gb5)],
                                o5_hbm.at[pl.ds(off, gb5)])
                return carry

            lax.fori_loop(0, (nblk5 - wid + 15) // 16, body5, 0)

        @pl.when(wid >= 16)
        def _():
            def body6(t, carry):
                off = (wid - 16 + t * 16) * gb6
                pltpu.sync_copy(a6_hbm.at[pl.ds(off, gb6)], i6)
                pltpu.async_copy(er_hbm.at[i6], rows, sem).wait()
                pltpu.sync_copy(rows, o6_hbm.at[pl.ds(off, gb6)])
                return carry

            lax.fori_loop(0, (nblk6 - (wid - 16) + 15) // 16, body6, 0)

    return k(edge_rep, a5, a6)


def _dot(x, w):
    return jnp.dot(x, w, preferred_element_type=jnp.float32)


def _iota16():
    return lax.iota(jnp.int32, 16)


def _zero_fill(zbuf, zr):
    def zv(t, carry):
        i = t // 8
        j = t - i * 8
        zbuf[i, pl.ds(j * 16, 16)] = jnp.zeros((16,), jnp.float32)
        return carry

    lax.fori_loop(0, zr * 8, zv, 0)


def _stage_atoms(ape_hbm, aidx, s):
    @pl.when(s < 8)
    def _():
        pltpu.sync_copy(ape_hbm.at[pl.ds(s * _TA5, _TA5)],
                        aidx.at[pl.ds(0, _TA5)])

    @pl.when(s >= 8)
    def _():
        pltpu.sync_copy(ape_hbm.at[pl.ds(_P5 + (s - 8) * _TA6, _TA6)],
                        aidx.at[pl.ds(0, _TA6)])


_LANE15 = None


def _splat_last(pc):
    return jnp.take(pc, jnp.full((16,), 15, jnp.int32))


def _compress(aidx, clist, plist, la, ta, tbase, srcoff, lo, hi, it16,
              double_out=False):
    junk = la - 16

    def cbody(j, cnt_v):
        v = aidx[pl.ds(j * 16, 16)]
        m = (v >= lo) & (v < hi)
        pc = plsc.cumsum(m.astype(jnp.int32))
        dest = jnp.where(m, cnt_v + pc - 1, junk + it16)
        cval = v - lo
        pos = tbase + j * 16 + it16 - srcoff
        if double_out:
            cval = cval + cval
            pos = pos + pos
        plsc.store_scatter(clist, [dest], cval)
        plsc.store_scatter(plist, [dest], pos)
        return cnt_v + _splat_last(pc)

    return lax.fori_loop(0, ta // 16, cbody, jnp.zeros((16,), jnp.int32))


def _refresh(dst, lst, b, gb, cnt_v, padbase, shift, it16):
    def cp(jj, carry):
        g = b * gb + jj * 16
        v = lst[pl.ds(g, 16)]
        pos16 = g + it16
        dst[pl.ds(jj * 16, 16)] = jnp.where(pos16 < cnt_v, v + shift,
                                            padbase + it16)
        return carry

    lax.fori_loop(0, gb // 16, cp, 0)


_CH1 = 10240
_GB1 = 128


def _edge_scatter_sc(lvl_all, ape, zeros):
    la = _TA6 + 2 * _GB1

    @functools.partial(
        pl.kernel,
        out_type=jax.ShapeDtypeStruct((E, H), jnp.float32),
        mesh=_SC_MESH,
        compiler_params=pltpu.CompilerParams(needs_layout_passes=False),
        scratch_types=[
            pltpu.VMEM((_TA6,), jnp.int32),
            pltpu.VMEM((la,), jnp.int32),
            pltpu.VMEM((la,), jnp.int32),
            pltpu.VMEM((_GB1,), jnp.int32),
            pltpu.VMEM((_GB1,), jnp.int32),
            pltpu.VMEM((_GB1, H), jnp.float32),
            pltpu.VMEM((40, H), jnp.float32),
            pltpu.VMEM_SHARED((_CH1 + 16, H), jnp.float32),
            pltpu.SemaphoreType.DMA,
        ],
    )
    def k(lvl_hbm, ape_hbm, z_hbm, out_hbm, aidx, clist, plist, cbuf, pbuf,
          rows, zbuf, acc, sem):
        c = lax.axis_index("c")
        s = lax.axis_index("s")
        it16 = _iota16()
        pltpu.sync_copy(z_hbm, zbuf)
        _stage_atoms(ape_hbm, aidx, s)
        ta = jnp.where(s < 8, _TA5, _TA6)
        tbase = jnp.where(s < 8, s * _TA5, _P5 + (s - 8) * _TA6)

        def one_pass(p, carry):
            lo = (c * 16 + p) * _CH1
            hi = lo + _CH1
            plsc.subcore_barrier()
            zd = [
                pltpu.async_copy(
                    zbuf, acc.at[pl.ds(s * 640 + r * 40, 40)], sem)
                for r in range(16)
            ]
            cnt_v = _compress(aidx, clist, plist, la, ta, tbase, 0,
                              lo, hi, it16)
            for d in zd:
                d.wait()
            plsc.subcore_barrier()

            def sbc(b):
                return jnp.any(cnt_v > b * _GB1)

            def sb(b):
                _refresh(cbuf, clist, b, _GB1, cnt_v, _CH1, 0, it16)
                _refresh(pbuf, plist, b, _GB1, cnt_v, 0, 0, it16)
                pltpu.async_copy(lvl_hbm.at[pbuf], rows, sem).wait()
                pltpu.sync_copy(rows, acc.at[cbuf], add=True)
                return b + 1

            lax.while_loop(sbc, sb, jnp.int32(0))
            plsc.subcore_barrier()

            @pl.when(lo + s * 640 < E)
            def _():
                pltpu.sync_copy(acc.at[pl.ds(s * 640, 640)],
                                out_hbm.at[pl.ds(lo + s * 640, 640)])

            return carry

        lax.fori_loop(0, 16, one_pass, 0)

    return k(lvl_all, ape, zeros)


_CH2 = 4800
_GB2 = 128


def _linmap_sc(cyc2, ape, zeros):
    la = _TA6 + 2 * _GB2

    @functools.partial(
        pl.kernel,
        out_type=jax.ShapeDtypeStruct((2 * _LROWS, H), jnp.float32),
        mesh=_SC_MESH,
        compiler_params=pltpu.CompilerParams(needs_layout_passes=False),
        scratch_types=[
            pltpu.VMEM((_TA6,), jnp.int32),
            pltpu.VMEM((la,), jnp.int32),
            pltpu.VMEM((la,), jnp.int32),
            pltpu.VMEM((_GB2,), jnp.int32),
            pltpu.VMEM((_GB2,), jnp.int32),
            pltpu.VMEM((_GB2, H), jnp.float32),
            pltpu.VMEM((40, H), jnp.float32),
            pltpu.VMEM_SHARED((2 * _CH2 + 16, H), jnp.float32),
            pltpu.SemaphoreType.DMA,
        ],
    )
    def k(cyc_hbm, ape_hbm, z_hbm, lin_hbm, aidx, clist, plist, cbuf, pbuf,
          rows, zbuf, acc, sem):
        c = lax.axis_index("c")
        s = lax.axis_index("s")
        it16 = _iota16()
        pltpu.sync_copy(z_hbm, zbuf)
        _stage_atoms(ape_hbm, aidx, s)
        ta = jnp.where(s < 8, _TA5, _TA6)
        tbase = jnp.where(s < 8, s * _TA5, _P5 + (s - 8) * _TA6)
        srccor = jnp.where(s < 8, 0, _P5 - 5 * C5)

        def one_pass(p, carry):
            lo = (c * 34 + p) * _CH2
            hi = lo + _CH2
            plsc.subcore_barrier()
            zd = [
                pltpu.async_copy(
                    zbuf, acc.at[pl.ds(s * 600 + r * 40, 40)], sem)
                for r in range(15)
            ]
            cnt_v = _compress(aidx, clist, plist, la, ta, tbase, srccor,
                              lo, hi, it16, double_out=True)
            for d in zd:
                d.wait()
            plsc.subcore_barrier()

            def sbc(b):
                return jnp.any(cnt_v > b * _GB2)

            def sb(b):
                for half in (0, 1):
                    _refresh(cbuf, clist, b, _GB2, cnt_v, 2 * _CH2 + half,
                             half, it16)
                    _refresh(pbuf, plist, b, _GB2, cnt_v, half, half, it16)
                    pltpu.async_copy(cyc_hbm.at[pbuf], rows, sem).wait()
                    pltpu.sync_copy(rows, acc.at[cbuf], add=True)
                return b + 1

            lax.while_loop(sbc, sb, jnp.int32(0))
            plsc.subcore_barrier()

            def sb2(b):
                for half in (0, 1):
                    _refresh(cbuf, clist, b, _GB2, cnt_v, 2 * _CH2 + half,
                             half, it16)
                    _refresh(pbuf, plist, b, _GB2, cnt_v, 2 * _AP + half,
                             srccor + srccor + half, it16)
                    pltpu.sync_copy(acc.at[cbuf], rows)
                    pltpu.sync_copy(rows, lin_hbm.at[pbuf])
                return b + 1

            lax.while_loop(sbc, sb2, jnp.int32(0))
            return carry

        lax.fori_loop(0, 34 - c, one_pass, 0)

    return k(cyc2, ape, zeros)


def _ne_body(lift_ref, er_ref, w1_ref, wl_ref, eps2_ref, emid_ref, eo1_ref):
    lift = lift_ref[...]
    w1 = w1_ref[...]
    e_mid = jnp.maximum(
        _dot(lift, w1[:H])
        + _dot(er_ref[...], w1[H:]),
        0.0,
    )
    emid_ref[...] = e_mid
    eo1_ref[...] = jnp.maximum(
        _dot((1.0 + eps2_ref[0, 0]) * e_mid + lift, wl_ref[...]),
        0.0,
    )


def _stage_ne(ne_lift, edge_rep, W_ne_lvl1, W_ne_lift, eps_ne_2):
    grid = E // _BE
    return pl.pallas_call(
        _ne_body,
        grid=(grid,),
        in_specs=[
            pl.BlockSpec((_BE, H), lambda i: (i, 0)),
            pl.BlockSpec((_BE, H), lambda i: (i, 0)),
            pl.BlockSpec((2 * H, H), lambda i: (0, 0)),
            pl.BlockSpec((H, H), lambda i: (0, 0)),
            pl.BlockSpec((1, 1), lambda i: (0, 0), memory_space=pltpu.SMEM),
        ],
        out_specs=[
            pl.BlockSpec((_BE, H), lambda i: (i, 0)),
            pl.BlockSpec((_BE, H), lambda i: (i, 0)),
        ],
        out_shape=[
            jax.ShapeDtypeStruct((E, H), jnp.float32),
            jax.ShapeDtypeStruct((E, H), jnp.float32),
        ],
    )(ne_lift, edge_rep, W_ne_lvl1, W_ne_lift, eps_ne_2.reshape(1, 1))


def _node_body(nr_ref, p0_ref, p1_ref, w_ref, eps_ref, out_ref):
    x = ((1.0 + eps_ref[0, 0]) * nr_ref[...] + p0_ref[0] + p1_ref[0])
    out_ref[...] = jnp.maximum(
        _dot(x, w_ref[...]), 0.0)


def _stage_node(node_rep, partials, W_ne_lvl2, eps_ne_1):
    bn = 2000
    return pl.pallas_call(
        _node_body,
        grid=(N // bn,),
        in_specs=[
            pl.BlockSpec((bn, H), lambda i: (i, 0)),
            pl.BlockSpec((1, bn, H), lambda i: (0, i, 0)),
            pl.BlockSpec((1, bn, H), lambda i: (1, i, 0)),
            pl.BlockSpec((H, H), lambda i: (0, 0)),
            pl.BlockSpec((1, 1), lambda i: (0, 0), memory_space=pltpu.SMEM),
        ],
        out_specs=pl.BlockSpec((bn, H), lambda i: (i, 0)),
        out_shape=jax.ShapeDtypeStruct((N, H), jnp.float32),
    )(node_rep, partials, partials, W_ne_lvl2, eps_ne_1.reshape(1, 1))


def _lvl1_body(k, e2c_ref, cyc_ref, w_ref, out_ref):
    bc = e2c_ref.shape[0] // k
    e2c = e2c_ref[...]
    s = jnp.sum(e2c.reshape(bc, k, H), axis=1)
    bsum = jnp.broadcast_to(s[:, None, :], (bc, k, H)).reshape(bc * k, H)
    w = w_ref[...]
    out_ref[...] = jnp.maximum(
        _dot(e2c, w[:H])
        + _dot(bsum, w[H:2 * H])
        + _dot(cyc_ref[...], w[2 * H:]),
        0.0,
    )


def _stage_lvl1(k, nc, e2c, cyc, W_ec_lvl1, off_blocks, buf=None):
    bc = 200
    rows = bc * k
    body = functools.partial(_lvl1_body, k)
    in_specs = [
        pl.BlockSpec((rows, H), lambda i: (i, 0)),
        pl.BlockSpec((rows, 2 * H), lambda i: (i, 0)),
        pl.BlockSpec((4 * H, H), lambda i: (0, 0)),
    ]
    args = [e2c, cyc, W_ec_lvl1]
    kwargs = {}
    if buf is not None:
        in_specs.append(pl.BlockSpec(memory_space=pl.MemorySpace.ANY))
        args.append(buf)
        kwargs["input_output_aliases"] = {3: 0}
        body = functools.partial(_lvl1_body_alias, k)
    return pl.pallas_call(
        body,
        grid=(nc // bc,),
        in_specs=in_specs,
        out_specs=pl.BlockSpec((rows, H),
                               lambda i: (off_blocks + i, 0)),
        out_shape=jax.ShapeDtypeStruct((_AP, H), jnp.float32),
        **kwargs,
    )(*args)


def _lvl1_body_alias(k, e2c_ref, cyc_ref, w_ref, buf_ref, out_ref):
    _lvl1_body(k, e2c_ref, cyc_ref, w_ref, out_ref)


def _eo_body(er_ref, lae_ref, eo1_ref, w2_ref, wm_ref, e11_ref, e12_ref,
             out_ref):
    eo2 = jnp.maximum(
        _dot((1.0 + e11_ref[0, 0]) * er_ref[...]
                + (1.0 + e12_ref[0, 0]) * lae_ref[...], w2_ref[...]),
        0.0,
    )
    wm = wm_ref[...]
    out_ref[...] = jnp.maximum(
        _dot(eo1_ref[...], wm[:H])
        + _dot(eo2, wm[H:]),
        0.0,
    )


def _stage_edge_out(edge_rep, lvl_aggr_e, edge_out_1, W_ec_lvl2, W_mlp,
                    eps_ec_11, eps_ec_12):
    return pl.pallas_call(
        _eo_body,
        grid=(E // _BE,),
        in_specs=[
            pl.BlockSpec((_BE, H), lambda i: (i, 0)),
            pl.BlockSpec((_BE, H), lambda i: (i, 0)),
            pl.BlockSpec((_BE, H), lambda i: (i, 0)),
            pl.BlockSpec((H, H), lambda i: (0, 0)),
            pl.BlockSpec((2 * H, H), lambda i: (0, 0)),
            pl.BlockSpec((1, 1), lambda i: (0, 0), memory_space=pltpu.SMEM),
            pl.BlockSpec((1, 1), lambda i: (0, 0), memory_space=pltpu.SMEM),
        ],
        out_specs=pl.BlockSpec((_BE, H), lambda i: (i, 0)),
        out_shape=jax.ShapeDtypeStruct((E, H), jnp.float32),
    )(edge_rep, lvl_aggr_e, edge_out_1, W_ec_lvl2, W_mlp,
      eps_ec_11.reshape(1, 1), eps_ec_12.reshape(1, 1))


def _cyc_body(k, lin_ref, e2c_ref, w_ref, eps_ref, out_ref):
    bc = e2c_ref.shape[0] // k
    e2c = e2c_ref[...]
    s = jnp.sum(e2c.reshape(bc, k, H), axis=1)
    bsum = jnp.broadcast_to(s[:, None, :], (bc, k, H)).reshape(bc * k, H)
    w = w_ref[...]
    out_ref[...] = jnp.maximum(
        (1.0 + eps_ref[0, 0])
        * _dot(lin_ref[...], w)
        + _dot(e2c, w[:H])
        + _dot(bsum, w[H:]),
        0.0,
    )


def _stage_cycle_out(k, nc, linmap, off_blocks, e2c, W_ec_lift, eps_ec_2):
    bc = 200
    rows = bc * k
    return pl.pallas_call(
        functools.partial(_cyc_body, k),
        grid=(nc // bc,),
        in_specs=[
            pl.BlockSpec((rows, 2 * H), lambda i: (off_blocks + i, 0)),
            pl.BlockSpec((rows, H), lambda i: (i, 0)),
            pl.BlockSpec((2 * H, 2 * H), lambda i: (0, 0)),
            pl.BlockSpec((1, 1), lambda i: (0, 0), memory_space=pltpu.SMEM),
        ],
        out_specs=pl.BlockSpec((rows, 2 * H), lambda i: (i, 0)),
        out_shape=jax.ShapeDtypeStruct((nc * k, 2 * H), jnp.float32),
    )(linmap, e2c, W_ec_lift, eps_ec_2.reshape(1, 1))


def kernel(node_rep, edge_rep, cycle_rep, edge_src, edge_dst, cycle5_edges,
           cycle6_edges, W_ne_lift, W_ne_lvl1, W_ne_lvl2, W_ec_lift,
           W_ec_lvl1, W_ec_lvl2, W_mlp, eps_ne_1, eps_ne_2, eps_ec_11,
           eps_ec_12, eps_ec_2):
    ne_lift = _ne_lift_sc(node_rep, edge_src, edge_dst)
    e_mid, edge_out_1 = _stage_ne(ne_lift, edge_rep, W_ne_lvl1, W_ne_lift,
                                  eps_ne_2)
    partials = _node_aggr_sc(e_mid, edge_src, edge_dst)
    node_out = _stage_node(node_rep, partials, W_ne_lvl2, eps_ne_1)

    a5 = cycle5_edges.reshape(-1)
    a6 = cycle6_edges.reshape(-1)
    e2c5, e2c6 = _e2c_sc(edge_rep, a5, a6)
    cyc5 = cycle_rep[:5 * C5]
    cyc6 = cycle_rep[5 * C5:]
    lvl_c5 = _stage_lvl1(5, C5, e2c5, cyc5, W_ec_lvl1, 0)
    lvl_all = _stage_lvl1(6, C6, e2c6, cyc6, W_ec_lvl1, _P5 // 1200,
                          buf=lvl_c5)

    sentinel = jnp.int32(1 << 20)
    ape = jnp.concatenate([
        a5, jnp.full((_P5 - 5 * C5,), sentinel, jnp.int32),
        a6, jnp.full((_AP - _P5 - 6 * C6,), sentinel, jnp.int32)])
    zeros40 = jnp.zeros((40, H), jnp.float32)
    lvl_aggr_e = _edge_scatter_sc(lvl_all, ape, zeros40)
    lin2 = _linmap_sc(cycle_rep.reshape(2 * A, H), ape, zeros40)
    linmap = lin2.reshape(_LROWS, 2 * H)

    edge_out = _stage_edge_out(edge_rep, lvl_aggr_e, edge_out_1, W_ec_lvl2,
                               W_mlp, eps_ec_11, eps_ec_12)
    co5 = _stage_cycle_out(5, C5, linmap, 0, e2c5, W_ec_lift, eps_ec_2)
    co6 = _stage_cycle_out(6, C6, linmap, _P5 // 1200, e2c6, W_ec_lift,
                           eps_ec_2)
    cycle_out = jnp.concatenate([co5, co6], axis=0)
    return (node_out, edge_out, cycle_out)


# speedup vs baseline: 1.0527x; 1.0357x over previous
import functools

import jax
import jax.numpy as jnp
from jax import lax
from jax.experimental import pallas as pl
from jax.experimental.pallas import tpu as pltpu
from jax.experimental.pallas import tpu_sc as plsc

N = 10000
E = 320000
H = 128
C5 = 10000
C6 = 10000
A = 5 * C5 + 6 * C6

_BE = 2000

_P5 = 57600
_AP = _P5 + 60032
_TA5 = _P5 // 8
_TA6 = (_AP - _P5) // 8
_LROWS = _AP + 16

_NC, _NS = 2, 16
_NW = _NC * _NS
_SC_MESH = plsc.VectorSubcoreMesh(core_axis_name="c", subcore_axis_name="s")


def _ne_lift_sc(node_rep, edge_src, edge_dst):
    epw = E // _NW
    gb = 200
    nb = epw // gb
    npr = 1000

    @functools.partial(
        pl.kernel,
        out_type=jax.ShapeDtypeStruct((E, H), jnp.float32),
        mesh=_SC_MESH,
        scratch_types=[
            pltpu.VMEM((gb,), jnp.int32),
            pltpu.VMEM((gb,), jnp.int32),
            pltpu.VMEM((gb, H), jnp.float32),
            pltpu.VMEM_SHARED((N, H), jnp.float32),
            pltpu.SemaphoreType.DMA,
            pltpu.SemaphoreType.DMA,
        ],
    )
    def k(node_hbm, src_hbm, dst_hbm, out_hbm, idx_s, idx_d, rows, nodes_sh,
          sem, sem2):
        c = lax.axis_index("c")
        s = lax.axis_index("s")
        wid = c * _NS + s

        @pl.when(s < N // npr)
        def _():
            pltpu.sync_copy(node_hbm.at[pl.ds(s * npr, npr)],
                            nodes_sh.at[pl.ds(s * npr, npr)])

        plsc.subcore_barrier()
        base = wid * epw

        def body(b, carry):
            off = base + b * gb
            d1 = pltpu.async_copy(src_hbm.at[pl.ds(off, gb)], idx_s, sem)
            d2 = pltpu.async_copy(dst_hbm.at[pl.ds(off, gb)], idx_d, sem2)
            d1.wait()
            d2.wait()
            pltpu.async_copy(nodes_sh.at[idx_s], rows, sem).wait()
            pltpu.async_copy(nodes_sh.at[idx_d], rows, sem, add=True).wait()
            pltpu.sync_copy(rows, out_hbm.at[pl.ds(off, gb)])
            return carry

        lax.fori_loop(0, nb, body, 0)

    return k(node_rep, edge_src, edge_dst)


def _node_aggr_sc(e_mid, edge_src, edge_dst):
    epw = E // _NW
    gb = 200
    nb = epw // gb
    npr = 1000
    zr = 40

    @functools.partial(
        pl.kernel,
        out_type=jax.ShapeDtypeStruct((_NC, N, H), jnp.float32),
        mesh=_SC_MESH,
        scratch_types=[
            pltpu.VMEM((gb,), jnp.int32),
            pltpu.VMEM((gb,), jnp.int32),
            pltpu.VMEM((gb, H), jnp.float32),
            pltpu.VMEM((zr, H), jnp.float32),
            pltpu.VMEM_SHARED((N, H), jnp.float32),
            pltpu.SemaphoreType.DMA,
            pltpu.SemaphoreType.DMA,
            pltpu.SemaphoreType.DMA,
        ],
    )
    def k(emid_hbm, src_hbm, dst_hbm, out_hbm, idx_s, idx_d, rows, zbuf, acc,
          sem, sem2, sem3):
        c = lax.axis_index("c")
        s = lax.axis_index("s")
        wid = c * _NS + s

        def zv(t, carry):
            i = t // 8
            j = t - i * 8
            zbuf[i, pl.ds(j * 16, 16)] = jnp.zeros((16,), jnp.float32)
            return carry

        lax.fori_loop(0, zr * 8, zv, 0)

        @pl.when(s < N // npr)
        def _():
            def zc(r, carry):
                pltpu.sync_copy(zbuf, acc.at[pl.ds(s * npr + r * zr, zr)])
                return carry

            lax.fori_loop(0, npr // zr, zc, 0)

        plsc.subcore_barrier()
        base = wid * epw

        def body(b, carry):
            off = base + b * gb
            d1 = pltpu.async_copy(src_hbm.at[pl.ds(off, gb)], idx_s, sem)
            d2 = pltpu.async_copy(dst_hbm.at[pl.ds(off, gb)], idx_d, sem2)
            d3 = pltpu.async_copy(emid_hbm.at[pl.ds(off, gb)], rows, sem3)
            d1.wait()
            d2.wait()
            d3.wait()
            d4 = pltpu.async_copy(rows, acc.at[idx_s], sem, add=True)
            d5 = pltpu.async_copy(rows, acc.at[idx_d], sem2, add=True)
            d4.wait()
            d5.wait()
            return carry

        lax.fori_loop(0, nb, body, 0)
        plsc.subcore_barrier()

        @pl.when(s < N // npr)
        def _():
            pltpu.sync_copy(acc.at[pl.ds(s * npr, npr)],
                            out_hbm.at[c, pl.ds(s * npr, npr)])

    return k(e_mid, edge_src, edge_dst)


def _e2c_sc(edge_rep, a5, a6):
    gb5, gb6 = 400, 480
    nblk5 = (5 * C5) // gb5
    nblk6 = (6 * C6) // gb6

    @functools.partial(
        pl.kernel,
        out_type=[
            jax.ShapeDtypeStruct((5 * C5, H), jnp.float32),
            jax.ShapeDtypeStruct((6 * C6, H), jnp.float32),
        ],
        mesh=_SC_MESH,
        scratch_types=[
            pltpu.VMEM((gb5,), jnp.int32),
            pltpu.VMEM((gb6,), jnp.int32),
            pltpu.VMEM((gb6, H), jnp.float32),
            pltpu.SemaphoreType.DMA,
        ],
    )
    def k(er_hbm, a5_hbm, a6_hbm, o5_hbm, o6_hbm, i5, i6, rows, sem):
        c = lax.axis_index("c")
        s = lax.axis_index("s")
        wid = c * _NS + s

        @pl.when(wid < 16)
        def _():
            def body5(t, carry):
                off = (wid + t * 16) * gb5
                pltpu.sync_copy(a5_hbm.at[pl.ds(off, gb5)], i5)
                pltpu.async_copy(er_hbm.at[i5], rows.at[pl.ds(0, gb5)],
                                 sem).wait()
                pltpu.sync_copy(rows.at[pl.ds(0, gb5)],
                                o5_hbm.at[pl.ds(off, gb5)])
                return carry

            lax.fori_loop(0, (nblk5 - wid + 15) // 16, body5, 0)

        @pl.when(wid >= 16)
        def _():
            def body6(t, carry):
                off = (wid - 16 + t * 16) * gb6
                pltpu.sync_copy(a6_hbm.at[pl.ds(off, gb6)], i6)
                pltpu.async_copy(er_hbm.at[i6], rows, sem).wait()
                pltpu.sync_copy(rows, o6_hbm.at[pl.ds(off, gb6)])
                return carry

            lax.fori_loop(0, (nblk6 - (wid - 16) + 15) // 16, body6, 0)

    return k(edge_rep, a5, a6)


def _dot(x, w):
    return jnp.dot(x, w, preferred_element_type=jnp.float32)


def _iota16():
    return lax.iota(jnp.int32, 16)


def _zero_fill(zbuf, zr):
    def zv(t, carry):
        i = t // 8
        j = t - i * 8
        zbuf[i, pl.ds(j * 16, 16)] = jnp.zeros((16,), jnp.float32)
        return carry

    lax.fori_loop(0, zr * 8, zv, 0)


def _stage_atoms(ape_hbm, aidx, s):
    @pl.when(s < 8)
    def _():
        pltpu.sync_copy(ape_hbm.at[pl.ds(s * _TA5, _TA5)],
                        aidx.at[pl.ds(0, _TA5)])

    @pl.when(s >= 8)
    def _():
        pltpu.sync_copy(ape_hbm.at[pl.ds(_P5 + (s - 8) * _TA6, _TA6)],
                        aidx.at[pl.ds(0, _TA6)])


_LANE15 = None


def _splat_last(pc):
    return jnp.take(pc, jnp.full((16,), 15, jnp.int32))


def _compress(aidx, clist, plist, la, ta, tbase, srcoff, lo, hi, it16,
              double_out=False):
    junk = la - 16

    def cbody(j, cnt_v):
        v = aidx[pl.ds(j * 16, 16)]
        m = (v >= lo) & (v < hi)
        pc = plsc.cumsum(m.astype(jnp.int32))
        dest = jnp.where(m, cnt_v + pc - 1, junk + it16)
        cval = v - lo
        pos = tbase + j * 16 + it16 - srcoff
        if double_out:
            cval = cval + cval
            pos = pos + pos
        plsc.store_scatter(clist, [dest], cval)
        plsc.store_scatter(plist, [dest], pos)
        return cnt_v + _splat_last(pc)

    return lax.fori_loop(0, ta // 16, cbody, jnp.zeros((16,), jnp.int32))


def _refresh(dst, lst, b, gb, cnt_v, padbase, shift, it16):
    def cp(jj, carry):
        g = b * gb + jj * 16
        v = lst[pl.ds(g, 16)]
        pos16 = g + it16
        dst[pl.ds(jj * 16, 16)] = jnp.where(pos16 < cnt_v, v + shift,
                                            padbase + it16)
        return carry

    lax.fori_loop(0, gb // 16, cp, 0)


_CH1 = 10240
_GB1 = 128


def _edge_scatter_sc(lvl_all, ape, zeros):
    la = _TA6 + 2 * _GB1

    @functools.partial(
        pl.kernel,
        out_type=jax.ShapeDtypeStruct((E, H), jnp.float32),
        mesh=_SC_MESH,
        compiler_params=pltpu.CompilerParams(needs_layout_passes=False),
        scratch_types=[
            pltpu.VMEM((_TA6,), jnp.int32),
            pltpu.VMEM((la,), jnp.int32),
            pltpu.VMEM((la,), jnp.int32),
            pltpu.VMEM((_GB1,), jnp.int32),
            pltpu.VMEM((_GB1,), jnp.int32),
            pltpu.VMEM((_GB1, H), jnp.float32),
            pltpu.VMEM((40, H), jnp.float32),
            pltpu.VMEM_SHARED((_CH1 + 16, H), jnp.float32),
            pltpu.SemaphoreType.DMA,
        ],
    )
    def k(lvl_hbm, ape_hbm, z_hbm, out_hbm, aidx, clist, plist, cbuf, pbuf,
          rows, zbuf, acc, sem):
        c = lax.axis_index("c")
        s = lax.axis_index("s")
        it16 = _iota16()
        pltpu.sync_copy(z_hbm, zbuf)
        _stage_atoms(ape_hbm, aidx, s)
        ta = jnp.where(s < 8, _TA5, _TA6)
        tbase = jnp.where(s < 8, s * _TA5, _P5 + (s - 8) * _TA6)

        def one_pass(p, carry):
            lo = (c * 16 + p) * _CH1
            hi = lo + _CH1
            plsc.subcore_barrier()
            zd = [
                pltpu.async_copy(
                    zbuf, acc.at[pl.ds(s * 640 + r * 40, 40)], sem)
                for r in range(16)
            ]
            cnt_v = _compress(aidx, clist, plist, la, ta, tbase, 0,
                              lo, hi, it16)
            for d in zd:
                d.wait()
            plsc.subcore_barrier()

            def sbc(b):
                return jnp.any(cnt_v > b * _GB1)

            def sb(b):
                _refresh(cbuf, clist, b, _GB1, cnt_v, _CH1, 0, it16)
                _refresh(pbuf, plist, b, _GB1, cnt_v, 0, 0, it16)
                pltpu.async_copy(lvl_hbm.at[pbuf], rows, sem).wait()
                pltpu.sync_copy(rows, acc.at[cbuf], add=True)
                return b + 1

            lax.while_loop(sbc, sb, jnp.int32(0))
            plsc.subcore_barrier()

            @pl.when(lo + s * 640 < E)
            def _():
                pltpu.sync_copy(acc.at[pl.ds(s * 640, 640)],
                                out_hbm.at[pl.ds(lo + s * 640, 640)])

            return carry

        lax.fori_loop(0, 16, one_pass, 0)

    return k(lvl_all, ape, zeros)


_CH2 = 4800
_GB2 = 128


def _linmap_sc(cyc2, ape, zeros):
    la = _TA6 + 2 * _GB2

    @functools.partial(
        pl.kernel,
        out_type=jax.ShapeDtypeStruct((2 * _LROWS, H), jnp.float32),
        mesh=_SC_MESH,
        compiler_params=pltpu.CompilerParams(needs_layout_passes=False),
        scratch_types=[
            pltpu.VMEM((_TA6,), jnp.int32),
            pltpu.VMEM((la,), jnp.int32),
            pltpu.VMEM((la,), jnp.int32),
            pltpu.VMEM((_GB2,), jnp.int32),
            pltpu.VMEM((_GB2,), jnp.int32),
            pltpu.VMEM((_GB2, H), jnp.float32),
            pltpu.VMEM((40, H), jnp.float32),
            pltpu.VMEM_SHARED((2 * _CH2 + 16, H), jnp.float32),
            pltpu.SemaphoreType.DMA,
        ],
    )
    def k(cyc_hbm, ape_hbm, z_hbm, lin_hbm, aidx, clist, plist, cbuf, pbuf,
          rows, zbuf, acc, sem):
        c = lax.axis_index("c")
        s = lax.axis_index("s")
        it16 = _iota16()
        pltpu.sync_copy(z_hbm, zbuf)
        _stage_atoms(ape_hbm, aidx, s)
        ta = jnp.where(s < 8, _TA5, _TA6)
        tbase = jnp.where(s < 8, s * _TA5, _P5 + (s - 8) * _TA6)
        srccor = jnp.where(s < 8, 0, _P5 - 5 * C5)

        def one_pass(p, carry):
            lo = (c * 34 + p) * _CH2
            hi = lo + _CH2
            plsc.subcore_barrier()
            zd = [
                pltpu.async_copy(
                    zbuf, acc.at[pl.ds(s * 600 + r * 40, 40)], sem)
                for r in range(15)
            ]
            cnt_v = _compress(aidx, clist, plist, la, ta, tbase, srccor,
                              lo, hi, it16, double_out=True)
            for d in zd:
                d.wait()
            plsc.subcore_barrier()

            def sbc(b):
                return jnp.any(cnt_v > b * _GB2)

            def sb(b):
                for half in (0, 1):
                    _refresh(cbuf, clist, b, _GB2, cnt_v, 2 * _CH2 + half,
                             half, it16)
                    _refresh(pbuf, plist, b, _GB2, cnt_v, half, half, it16)
                    pltpu.async_copy(cyc_hbm.at[pbuf], rows, sem).wait()
                    pltpu.sync_copy(rows, acc.at[cbuf], add=True)
                return b + 1

            lax.while_loop(sbc, sb, jnp.int32(0))
            plsc.subcore_barrier()

            def sb2(b):
                for half in (0, 1):
                    _refresh(cbuf, clist, b, _GB2, cnt_v, 2 * _CH2 + half,
                             half, it16)
                    _refresh(pbuf, plist, b, _GB2, cnt_v, 2 * _AP + half,
                             srccor + srccor + half, it16)
                    pltpu.sync_copy(acc.at[cbuf], rows)
                    pltpu.sync_copy(rows, lin_hbm.at[pbuf])
                return b + 1

            lax.while_loop(sbc, sb2, jnp.int32(0))
            return carry

        lax.fori_loop(0, 34 - c, one_pass, 0)

    return k(cyc2, ape, zeros)


def _ne_body(lift_ref, er_ref, w1_ref, wl_ref, eps2_ref, emid_ref, eo1_ref):
    lift = lift_ref[...]
    w1 = w1_ref[...]
    e_mid = jnp.maximum(
        _dot(lift, w1[:H])
        + _dot(er_ref[...], w1[H:]),
        0.0,
    )
    emid_ref[...] = e_mid
    eo1_ref[...] = jnp.maximum(
        _dot((1.0 + eps2_ref[0, 0]) * e_mid + lift, wl_ref[...]),
        0.0,
    )


def _stage_ne(ne_lift, edge_rep, W_ne_lvl1, W_ne_lift, eps_ne_2):
    grid = E // _BE
    return pl.pallas_call(
        _ne_body,
        grid=(grid,),
        in_specs=[
            pl.BlockSpec((_BE, H), lambda i: (i, 0)),
            pl.BlockSpec((_BE, H), lambda i: (i, 0)),
            pl.BlockSpec((2 * H, H), lambda i: (0, 0)),
            pl.BlockSpec((H, H), lambda i: (0, 0)),
            pl.BlockSpec((1, 1), lambda i: (0, 0), memory_space=pltpu.SMEM),
        ],
        out_specs=[
            pl.BlockSpec((_BE, H), lambda i: (i, 0)),
            pl.BlockSpec((_BE, H), lambda i: (i, 0)),
        ],
        out_shape=[
            jax.ShapeDtypeStruct((E, H), jnp.float32),
            jax.ShapeDtypeStruct((E, H), jnp.float32),
        ],
    )(ne_lift, edge_rep, W_ne_lvl1, W_ne_lift, eps_ne_2.reshape(1, 1))


def _node_body(nr_ref, p0_ref, p1_ref, w_ref, eps_ref, out_ref):
    x = ((1.0 + eps_ref[0, 0]) * nr_ref[...] + p0_ref[0] + p1_ref[0])
    out_ref[...] = jnp.maximum(
        _dot(x, w_ref[...]), 0.0)


def _stage_node(node_rep, partials, W_ne_lvl2, eps_ne_1):
    bn = 2000
    return pl.pallas_call(
        _node_body,
        grid=(N // bn,),
        in_specs=[
            pl.BlockSpec((bn, H), lambda i: (i, 0)),
            pl.BlockSpec((1, bn, H), lambda i: (0, i, 0)),
            pl.BlockSpec((1, bn, H), lambda i: (1, i, 0)),
            pl.BlockSpec((H, H), lambda i: (0, 0)),
            pl.BlockSpec((1, 1), lambda i: (0, 0), memory_space=pltpu.SMEM),
        ],
        out_specs=pl.BlockSpec((bn, H), lambda i: (i, 0)),
        out_shape=jax.ShapeDtypeStruct((N, H), jnp.float32),
    )(node_rep, partials, partials, W_ne_lvl2, eps_ne_1.reshape(1, 1))


def _lvl1_body(k, e2c_ref, cyc_ref, w_ref, out_ref):
    bc = e2c_ref.shape[0] // k
    e2c = e2c_ref[...]
    s = jnp.sum(e2c.reshape(bc, k, H), axis=1)
    bsum = jnp.broadcast_to(s[:, None, :], (bc, k, H)).reshape(bc * k, H)
    w = w_ref[...]
    out_ref[...] = jnp.maximum(
        _dot(e2c, w[:H])
        + _dot(bsum, w[H:2 * H])
        + _dot(cyc_ref[...], w[2 * H:]),
        0.0,
    )


def _stage_lvl1(k, nc, e2c, cyc, W_ec_lvl1, off_blocks, buf=None):
    bc = 200
    rows = bc * k
    body = functools.partial(_lvl1_body, k)
    in_specs = [
        pl.BlockSpec((rows, H), lambda i: (i, 0)),
        pl.BlockSpec((rows, 2 * H), lambda i: (i, 0)),
        pl.BlockSpec((4 * H, H), lambda i: (0, 0)),
    ]
    args = [e2c, cyc, W_ec_lvl1]
    kwargs = {}
    if buf is not None:
        in_specs.append(pl.BlockSpec(memory_space=pl.MemorySpace.ANY))
        args.append(buf)
        kwargs["input_output_aliases"] = {3: 0}
        body = functools.partial(_lvl1_body_alias, k)
    return pl.pallas_call(
        body,
        grid=(nc // bc,),
        in_specs=in_specs,
        out_specs=pl.BlockSpec((rows, H),
                               lambda i: (off_blocks + i, 0)),
        out_shape=jax.ShapeDtypeStruct((_AP, H), jnp.float32),
        **kwargs,
    )(*args)


def _lvl1_body_alias(k, e2c_ref, cyc_ref, w_ref, buf_ref, out_ref):
    _lvl1_body(k, e2c_ref, cyc_ref, w_ref, out_ref)


def _eo_body(er_ref, lae_ref, eo1_ref, w2_ref, wm_ref, e11_ref, e12_ref,
             out_ref):
    eo2 = jnp.maximum(
        _dot((1.0 + e11_ref[0, 0]) * er_ref[...]
                + (1.0 + e12_ref[0, 0]) * lae_ref[...], w2_ref[...]),
        0.0,
    )
    wm = wm_ref[...]
    out_ref[...] = jnp.maximum(
        _dot(eo1_ref[...], wm[:H])
        + _dot(eo2, wm[H:]),
        0.0,
    )


def _stage_edge_out(edge_rep, lvl_aggr_e, edge_out_1, W_ec_lvl2, W_mlp,
                    eps_ec_11, eps_ec_12):
    return pl.pallas_call(
        _eo_body,
        grid=(E // _BE,),
        in_specs=[
            pl.BlockSpec((_BE, H), lambda i: (i, 0)),
            pl.BlockSpec((_BE, H), lambda i: (i, 0)),
            pl.BlockSpec((_BE, H), lambda i: (i, 0)),
            pl.BlockSpec((H, H), lambda i: (0, 0)),
            pl.BlockSpec((2 * H, H), lambda i: (0, 0)),
            pl.BlockSpec((1, 1), lambda i: (0, 0), memory_space=pltpu.SMEM),
            pl.BlockSpec((1, 1), lambda i: (0, 0), memory_space=pltpu.SMEM),
        ],
        out_specs=pl.BlockSpec((_BE, H), lambda i: (i, 0)),
        out_shape=jax.ShapeDtypeStruct((E, H), jnp.float32),
    )(edge_rep, lvl_aggr_e, edge_out_1, W_ec_lvl2, W_mlp,
      eps_ec_11.reshape(1, 1), eps_ec_12.reshape(1, 1))


def _cyc_body(k, lin_ref, e2c_ref, w_ref, eps_ref, out_ref):
    bc = e2c_ref.shape[0] // k
    e2c = e2c_ref[...]
    s = jnp.sum(e2c.reshape(bc, k, H), axis=1)
    bsum = jnp.broadcast_to(s[:, None, :], (bc, k, H)).reshape(bc * k, H)
    w = w_ref[...]
    out_ref[...] = jnp.maximum(
        (1.0 + eps_ref[0, 0])
        * _dot(lin_ref[...], w)
        + _dot(e2c, w[:H])
        + _dot(bsum, w[H:]),
        0.0,
    )


def _stage_cycle_out(k, nc, linmap, off_blocks, e2c, W_ec_lift, eps_ec_2):
    bc = 200
    rows = bc * k
    return pl.pallas_call(
        functools.partial(_cyc_body, k),
        grid=(nc // bc,),
        in_specs=[
            pl.BlockSpec((rows, 2 * H), lambda i: (off_blocks + i, 0)),
            pl.BlockSpec((rows, H), lambda i: (i, 0)),
            pl.BlockSpec((2 * H, 2 * H), lambda i: (0, 0)),
            pl.BlockSpec((1, 1), lambda i: (0, 0), memory_space=pltpu.SMEM),
        ],
        out_specs=pl.BlockSpec((rows, 2 * H), lambda i: (i, 0)),
        out_shape=jax.ShapeDtypeStruct((nc * k, 2 * H), jnp.float32),
    )(linmap, e2c, W_ec_lift, eps_ec_2.reshape(1, 1))


def kernel(node_rep, edge_rep, cycle_rep, edge_src, edge_dst, cycle5_edges,
           cycle6_edges, W_ne_lift, W_ne_lvl1, W_ne_lvl2, W_ec_lift,
           W_ec_lvl1, W_ec_lvl2, W_mlp, eps_ne_1, eps_ne_2, eps_ec_11,
           eps_ec_12, eps_ec_2):
    ne_lift = _ne_lift_sc(node_rep, edge_src, edge_dst)
    e_mid, edge_out_1 = _stage_ne(ne_lift, edge_rep, W_ne_lvl1, W_ne_lift,
                                  eps_ne_2)

    a5 = cycle5_edges.reshape(-1)
    a6 = cycle6_edges.reshape(-1)
    e2c5, e2c6 = _e2c_sc(edge_rep, a5, a6)
    cyc5 = cycle_rep[:5 * C5]
    cyc6 = cycle_rep[5 * C5:]
    lvl_c5 = _stage_lvl1(5, C5, e2c5, cyc5, W_ec_lvl1, 0)
    lvl_all = _stage_lvl1(6, C6, e2c6, cyc6, W_ec_lvl1, _P5 // 1200,
                          buf=lvl_c5)

    sentinel = jnp.int32(1 << 20)
    ape = jnp.concatenate([
        a5, jnp.full((_P5 - 5 * C5,), sentinel, jnp.int32),
        a6, jnp.full((_AP - _P5 - 6 * C6,), sentinel, jnp.int32)])
    zeros40 = jnp.zeros((40, H), jnp.float32)
    lvl_aggr_e = _edge_scatter_sc(lvl_all, ape, zeros40)
    lin2 = _linmap_sc(cycle_rep.reshape(2 * A, H), ape, zeros40)
    linmap = lin2.reshape(_LROWS, 2 * H)

    edge_out = _stage_edge_out(edge_rep, lvl_aggr_e, edge_out_1, W_ec_lvl2,
                               W_mlp, eps_ec_11, eps_ec_12)
    co5 = _stage_cycle_out(5, C5, linmap, 0, e2c5, W_ec_lift, eps_ec_2)
    co6 = _stage_cycle_out(6, C6, linmap, _P5 // 1200, e2c6, W_ec_lift,
                           eps_ec_2)
    cycle_out = jnp.concatenate([co5, co6], axis=0)

    partials = _node_aggr_sc(e_mid, edge_src, edge_dst)
    node_out = _stage_node(node_rep, partials, W_ne_lvl2, eps_ne_1)
    return (node_out, edge_out, cycle_out)


# speedup vs baseline: 1.0777x; 1.0238x over previous
import functools

import jax
import jax.numpy as jnp
from jax import lax
from jax.experimental import pallas as pl
from jax.experimental.pallas import tpu as pltpu
from jax.experimental.pallas import tpu_sc as plsc

N = 10000
E = 320000
H = 128
C5 = 10000
C6 = 10000
A = 5 * C5 + 6 * C6

_BE = 4000

_P5 = 57600
_AP = _P5 + 60032
_TA5 = _P5 // 8
_TA6 = (_AP - _P5) // 8
_LROWS = _AP + 16

_NC, _NS = 2, 16
_NW = _NC * _NS
_SC_MESH = plsc.VectorSubcoreMesh(core_axis_name="c", subcore_axis_name="s")


def _ne_lift_sc(node_rep, edge_src, edge_dst):
    epw = E // _NW
    gb = 200
    nb = epw // gb
    npr = 1000

    @functools.partial(
        pl.kernel,
        out_type=jax.ShapeDtypeStruct((E, H), jnp.float32),
        mesh=_SC_MESH,
        scratch_types=[
            pltpu.VMEM((gb,), jnp.int32),
            pltpu.VMEM((gb,), jnp.int32),
            pltpu.VMEM((gb, H), jnp.float32),
            pltpu.VMEM_SHARED((N, H), jnp.float32),
            pltpu.SemaphoreType.DMA,
            pltpu.SemaphoreType.DMA,
        ],
    )
    def k(node_hbm, src_hbm, dst_hbm, out_hbm, idx_s, idx_d, rows, nodes_sh,
          sem, sem2):
        c = lax.axis_index("c")
        s = lax.axis_index("s")
        wid = c * _NS + s

        @pl.when(s < N // npr)
        def _():
            pltpu.sync_copy(node_hbm.at[pl.ds(s * npr, npr)],
                            nodes_sh.at[pl.ds(s * npr, npr)])

        plsc.subcore_barrier()
        base = wid * epw

        def body(b, carry):
            off = base + b * gb
            d1 = pltpu.async_copy(src_hbm.at[pl.ds(off, gb)], idx_s, sem)
            d2 = pltpu.async_copy(dst_hbm.at[pl.ds(off, gb)], idx_d, sem2)
            d1.wait()
            d2.wait()
            pltpu.async_copy(nodes_sh.at[idx_s], rows, sem).wait()
            pltpu.async_copy(nodes_sh.at[idx_d], rows, sem, add=True).wait()
            pltpu.sync_copy(rows, out_hbm.at[pl.ds(off, gb)])
            return carry

        lax.fori_loop(0, nb, body, 0)

    return k(node_rep, edge_src, edge_dst)


def _node_aggr_sc(e_mid, edge_src, edge_dst):
    epw = E // _NW
    gb = 200
    nb = epw // gb
    npr = 1000
    zr = 40

    @functools.partial(
        pl.kernel,
        out_type=jax.ShapeDtypeStruct((_NC, N, H), jnp.float32),
        mesh=_SC_MESH,
        scratch_types=[
            pltpu.VMEM((gb,), jnp.int32),
            pltpu.VMEM((gb,), jnp.int32),
            pltpu.VMEM((gb, H), jnp.float32),
            pltpu.VMEM((zr, H), jnp.float32),
            pltpu.VMEM_SHARED((N, H), jnp.float32),
            pltpu.SemaphoreType.DMA,
            pltpu.SemaphoreType.DMA,
            pltpu.SemaphoreType.DMA,
        ],
    )
    def k(emid_hbm, src_hbm, dst_hbm, out_hbm, idx_s, idx_d, rows, zbuf, acc,
          sem, sem2, sem3):
        c = lax.axis_index("c")
        s = lax.axis_index("s")
        wid = c * _NS + s

        def zv(t, carry):
            i = t // 8
            j = t - i * 8
            zbuf[i, pl.ds(j * 16, 16)] = jnp.zeros((16,), jnp.float32)
            return carry

        lax.fori_loop(0, zr * 8, zv, 0)

        @pl.when(s < N // npr)
        def _():
            def zc(r, carry):
                pltpu.sync_copy(zbuf, acc.at[pl.ds(s * npr + r * zr, zr)])
                return carry

            lax.fori_loop(0, npr // zr, zc, 0)

        plsc.subcore_barrier()
        base = wid * epw

        def body(b, carry):
            off = base + b * gb
            d1 = pltpu.async_copy(src_hbm.at[pl.ds(off, gb)], idx_s, sem)
            d2 = pltpu.async_copy(dst_hbm.at[pl.ds(off, gb)], idx_d, sem2)
            d3 = pltpu.async_copy(emid_hbm.at[pl.ds(off, gb)], rows, sem3)
            d1.wait()
            d2.wait()
            d3.wait()
            d4 = pltpu.async_copy(rows, acc.at[idx_s], sem, add=True)
            d5 = pltpu.async_copy(rows, acc.at[idx_d], sem2, add=True)
            d4.wait()
            d5.wait()
            return carry

        lax.fori_loop(0, nb, body, 0)
        plsc.subcore_barrier()

        @pl.when(s < N // npr)
        def _():
            pltpu.sync_copy(acc.at[pl.ds(s * npr, npr)],
                            out_hbm.at[c, pl.ds(s * npr, npr)])

    return k(e_mid, edge_src, edge_dst)


def _e2c_sc(edge_rep, a5, a6):
    gb5, gb6 = 400, 480
    nblk5 = (5 * C5) // gb5
    nblk6 = (6 * C6) // gb6

    @functools.partial(
        pl.kernel,
        out_type=[
            jax.ShapeDtypeStruct((5 * C5, H), jnp.float32),
            jax.ShapeDtypeStruct((6 * C6, H), jnp.float32),
        ],
        mesh=_SC_MESH,
        scratch_types=[
            pltpu.VMEM((gb5,), jnp.int32),
            pltpu.VMEM((gb6,), jnp.int32),
            pltpu.VMEM((gb6, H), jnp.float32),
            pltpu.SemaphoreType.DMA,
        ],
    )
    def k(er_hbm, a5_hbm, a6_hbm, o5_hbm, o6_hbm, i5, i6, rows, sem):
        c = lax.axis_index("c")
        s = lax.axis_index("s")
        wid = c * _NS + s

        @pl.when(wid < 16)
        def _():
            def body5(t, carry):
                off = (wid + t * 16) * gb5
                pltpu.sync_copy(a5_hbm.at[pl.ds(off, gb5)], i5)
                pltpu.async_copy(er_hbm.at[i5], rows.at[pl.ds(0, gb5)],
                                 sem).wait()
                pltpu.sync_copy(rows.at[pl.ds(0, gb5)],
                                o5_hbm.at[pl.ds(off, gb5)])
                return carry

            lax.fori_loop(0, (nblk5 - wid + 15) // 16, body5, 0)

        @pl.when(wid >= 16)
        def _():
            def body6(t, carry):
                off = (wid - 16 + t * 16) * gb6
                pltpu.sync_copy(a6_hbm.at[pl.ds(off, gb6)], i6)
                pltpu.async_copy(er_hbm.at[i6], rows, sem).wait()
                pltpu.sync_copy(rows, o6_hbm.at[pl.ds(off, gb6)])
                return carry

            lax.fori_loop(0, (nblk6 - (wid - 16) + 15) // 16, body6, 0)

    return k(edge_rep, a5, a6)


def _dot(x, w):
    return jnp.dot(x, w, preferred_element_type=jnp.float32)


def _iota16():
    return lax.iota(jnp.int32, 16)


def _zero_fill(zbuf, zr):
    def zv(t, carry):
        i = t // 8
        j = t - i * 8
        zbuf[i, pl.ds(j * 16, 16)] = jnp.zeros((16,), jnp.float32)
        return carry

    lax.fori_loop(0, zr * 8, zv, 0)


def _stage_atoms(ape_hbm, aidx, s):
    @pl.when(s < 8)
    def _():
        pltpu.sync_copy(ape_hbm.at[pl.ds(s * _TA5, _TA5)],
                        aidx.at[pl.ds(0, _TA5)])

    @pl.when(s >= 8)
    def _():
        pltpu.sync_copy(ape_hbm.at[pl.ds(_P5 + (s - 8) * _TA6, _TA6)],
                        aidx.at[pl.ds(0, _TA6)])


_LANE15 = None


def _splat_last(pc):
    return jnp.take(pc, jnp.full((16,), 15, jnp.int32))


def _compress(aidx, clist, plist, la, ta, tbase, srcoff, lo, hi, it16,
              double_out=False):
    junk = la - 16

    def cbody(j, cnt_v):
        v = aidx[pl.ds(j * 16, 16)]
        m = (v >= lo) & (v < hi)
        pc = plsc.cumsum(m.astype(jnp.int32))
        dest = jnp.where(m, cnt_v + pc - 1, junk + it16)
        cval = v - lo
        pos = tbase + j * 16 + it16 - srcoff
        if double_out:
            cval = cval + cval
            pos = pos + pos
        plsc.store_scatter(clist, [dest], cval)
        plsc.store_scatter(plist, [dest], pos)
        return cnt_v + _splat_last(pc)

    return lax.fori_loop(0, ta // 16, cbody, jnp.zeros((16,), jnp.int32))


def _refresh(dst, lst, b, gb, cnt_v, padbase, shift, it16):
    def cp(jj, carry):
        g = b * gb + jj * 16
        v = lst[pl.ds(g, 16)]
        pos16 = g + it16
        dst[pl.ds(jj * 16, 16)] = jnp.where(pos16 < cnt_v, v + shift,
                                            padbase + it16)
        return carry

    lax.fori_loop(0, gb // 16, cp, 0)


_CH1 = 10240
_GB1 = 128


def _edge_scatter_sc(lvl_all, ape, zeros):
    la = _TA6 + 2 * _GB1

    @functools.partial(
        pl.kernel,
        out_type=jax.ShapeDtypeStruct((E, H), jnp.float32),
        mesh=_SC_MESH,
        compiler_params=pltpu.CompilerParams(needs_layout_passes=False),
        scratch_types=[
            pltpu.VMEM((_TA6,), jnp.int32),
            pltpu.VMEM((la,), jnp.int32),
            pltpu.VMEM((la,), jnp.int32),
            pltpu.VMEM((_GB1,), jnp.int32),
            pltpu.VMEM((_GB1,), jnp.int32),
            pltpu.VMEM((_GB1, H), jnp.float32),
            pltpu.VMEM((40, H), jnp.float32),
            pltpu.VMEM_SHARED((_CH1 + 16, H), jnp.float32),
            pltpu.SemaphoreType.DMA,
        ],
    )
    def k(lvl_hbm, ape_hbm, z_hbm, out_hbm, aidx, clist, plist, cbuf, pbuf,
          rows, zbuf, acc, sem):
        c = lax.axis_index("c")
        s = lax.axis_index("s")
        it16 = _iota16()
        pltpu.sync_copy(z_hbm, zbuf)
        _stage_atoms(ape_hbm, aidx, s)
        ta = jnp.where(s < 8, _TA5, _TA6)
        tbase = jnp.where(s < 8, s * _TA5, _P5 + (s - 8) * _TA6)

        def one_pass(p, carry):
            lo = (c * 16 + p) * _CH1
            hi = lo + _CH1
            plsc.subcore_barrier()
            zd = [
                pltpu.async_copy(
                    zbuf, acc.at[pl.ds(s * 640 + r * 40, 40)], sem)
                for r in range(16)
            ]
            cnt_v = _compress(aidx, clist, plist, la, ta, tbase, 0,
                              lo, hi, it16)
            for d in zd:
                d.wait()
            plsc.subcore_barrier()

            def sbc(b):
                return jnp.any(cnt_v > b * _GB1)

            def sb(b):
                _refresh(cbuf, clist, b, _GB1, cnt_v, _CH1, 0, it16)
                _refresh(pbuf, plist, b, _GB1, cnt_v, 0, 0, it16)
                pltpu.async_copy(lvl_hbm.at[pbuf], rows, sem).wait()
                pltpu.sync_copy(rows, acc.at[cbuf], add=True)
                return b + 1

            lax.while_loop(sbc, sb, jnp.int32(0))
            plsc.subcore_barrier()

            @pl.when(lo + s * 640 < E)
            def _():
                pltpu.sync_copy(acc.at[pl.ds(s * 640, 640)],
                                out_hbm.at[pl.ds(lo + s * 640, 640)])

            return carry

        lax.fori_loop(0, 16, one_pass, 0)

    return k(lvl_all, ape, zeros)


_CH2 = 4800
_GB2 = 128


def _linmap_sc(cyc2, ape, zeros):
    la = _TA6 + 2 * _GB2

    @functools.partial(
        pl.kernel,
        out_type=jax.ShapeDtypeStruct((2 * _LROWS, H), jnp.float32),
        mesh=_SC_MESH,
        compiler_params=pltpu.CompilerParams(needs_layout_passes=False),
        scratch_types=[
            pltpu.VMEM((_TA6,), jnp.int32),
            pltpu.VMEM((la,), jnp.int32),
            pltpu.VMEM((la,), jnp.int32),
            pltpu.VMEM((_GB2,), jnp.int32),
            pltpu.VMEM((_GB2,), jnp.int32),
            pltpu.VMEM((_GB2, H), jnp.float32),
            pltpu.VMEM((40, H), jnp.float32),
            pltpu.VMEM_SHARED((2 * _CH2 + 16, H), jnp.float32),
            pltpu.SemaphoreType.DMA,
        ],
    )
    def k(cyc_hbm, ape_hbm, z_hbm, lin_hbm, aidx, clist, plist, cbuf, pbuf,
          rows, zbuf, acc, sem):
        c = lax.axis_index("c")
        s = lax.axis_index("s")
        it16 = _iota16()
        pltpu.sync_copy(z_hbm, zbuf)
        _stage_atoms(ape_hbm, aidx, s)
        ta = jnp.where(s < 8, _TA5, _TA6)
        tbase = jnp.where(s < 8, s * _TA5, _P5 + (s - 8) * _TA6)
        srccor = jnp.where(s < 8, 0, _P5 - 5 * C5)

        def one_pass(p, carry):
            lo = (c * 34 + p) * _CH2
            hi = lo + _CH2
            plsc.subcore_barrier()
            zd = [
                pltpu.async_copy(
                    zbuf, acc.at[pl.ds(s * 600 + r * 40, 40)], sem)
                for r in range(15)
            ]
            cnt_v = _compress(aidx, clist, plist, la, ta, tbase, srccor,
                              lo, hi, it16, double_out=True)
            for d in zd:
                d.wait()
            plsc.subcore_barrier()

            def sbc(b):
                return jnp.any(cnt_v > b * _GB2)

            def sb(b):
                for half in (0, 1):
                    _refresh(cbuf, clist, b, _GB2, cnt_v, 2 * _CH2 + half,
                             half, it16)
                    _refresh(pbuf, plist, b, _GB2, cnt_v, half, half, it16)
                    pltpu.async_copy(cyc_hbm.at[pbuf], rows, sem).wait()
                    pltpu.sync_copy(rows, acc.at[cbuf], add=True)
                return b + 1

            lax.while_loop(sbc, sb, jnp.int32(0))
            plsc.subcore_barrier()

            def sb2(b):
                for half in (0, 1):
                    _refresh(cbuf, clist, b, _GB2, cnt_v, 2 * _CH2 + half,
                             half, it16)
                    _refresh(pbuf, plist, b, _GB2, cnt_v, 2 * _AP + half,
                             srccor + srccor + half, it16)
                    pltpu.sync_copy(acc.at[cbuf], rows)
                    pltpu.sync_copy(rows, lin_hbm.at[pbuf])
                return b + 1

            lax.while_loop(sbc, sb2, jnp.int32(0))
            return carry

        lax.fori_loop(0, 34 - c, one_pass, 0)

    return k(cyc2, ape, zeros)


def _ne_body(lift_ref, er_ref, w1_ref, wl_ref, eps2_ref, emid_ref, eo1_ref):
    lift = lift_ref[...]
    w1 = w1_ref[...]
    e_mid = jnp.maximum(
        _dot(lift, w1[:H])
        + _dot(er_ref[...], w1[H:]),
        0.0,
    )
    emid_ref[...] = e_mid
    eo1_ref[...] = jnp.maximum(
        _dot((1.0 + eps2_ref[0, 0]) * e_mid + lift, wl_ref[...]),
        0.0,
    )


def _stage_ne(ne_lift, edge_rep, W_ne_lvl1, W_ne_lift, eps_ne_2):
    grid = E // _BE
    return pl.pallas_call(
        _ne_body,
        grid=(grid,),
        in_specs=[
            pl.BlockSpec((_BE, H), lambda i: (i, 0)),
            pl.BlockSpec((_BE, H), lambda i: (i, 0)),
            pl.BlockSpec((2 * H, H), lambda i: (0, 0)),
            pl.BlockSpec((H, H), lambda i: (0, 0)),
            pl.BlockSpec((1, 1), lambda i: (0, 0), memory_space=pltpu.SMEM),
        ],
        out_specs=[
            pl.BlockSpec((_BE, H), lambda i: (i, 0)),
            pl.BlockSpec((_BE, H), lambda i: (i, 0)),
        ],
        out_shape=[
            jax.ShapeDtypeStruct((E, H), jnp.float32),
            jax.ShapeDtypeStruct((E, H), jnp.float32),
        ],
    )(ne_lift, edge_rep, W_ne_lvl1, W_ne_lift, eps_ne_2.reshape(1, 1))


def _node_body(nr_ref, p0_ref, p1_ref, w_ref, eps_ref, out_ref):
    x = ((1.0 + eps_ref[0, 0]) * nr_ref[...] + p0_ref[0] + p1_ref[0])
    out_ref[...] = jnp.maximum(
        _dot(x, w_ref[...]), 0.0)


def _stage_node(node_rep, partials, W_ne_lvl2, eps_ne_1):
    bn = 2000
    return pl.pallas_call(
        _node_body,
        grid=(N // bn,),
        in_specs=[
            pl.BlockSpec((bn, H), lambda i: (i, 0)),
            pl.BlockSpec((1, bn, H), lambda i: (0, i, 0)),
            pl.BlockSpec((1, bn, H), lambda i: (1, i, 0)),
            pl.BlockSpec((H, H), lambda i: (0, 0)),
            pl.BlockSpec((1, 1), lambda i: (0, 0), memory_space=pltpu.SMEM),
        ],
        out_specs=pl.BlockSpec((bn, H), lambda i: (i, 0)),
        out_shape=jax.ShapeDtypeStruct((N, H), jnp.float32),
    )(node_rep, partials, partials, W_ne_lvl2, eps_ne_1.reshape(1, 1))


def _lvl1_body(k, e2c_ref, cyc_ref, w_ref, out_ref):
    bc = e2c_ref.shape[0] // k
    e2c = e2c_ref[...]
    s = jnp.sum(e2c.reshape(bc, k, H), axis=1)
    bsum = jnp.broadcast_to(s[:, None, :], (bc, k, H)).reshape(bc * k, H)
    w = w_ref[...]
    out_ref[...] = jnp.maximum(
        _dot(e2c, w[:H])
        + _dot(bsum, w[H:2 * H])
        + _dot(cyc_ref[...], w[2 * H:]),
        0.0,
    )


def _stage_lvl1(k, nc, e2c, cyc, W_ec_lvl1, off_blocks, buf=None):
    bc = 200
    rows = bc * k
    body = functools.partial(_lvl1_body, k)
    in_specs = [
        pl.BlockSpec((rows, H), lambda i: (i, 0)),
        pl.BlockSpec((rows, 2 * H), lambda i: (i, 0)),
        pl.BlockSpec((4 * H, H), lambda i: (0, 0)),
    ]
    args = [e2c, cyc, W_ec_lvl1]
    kwargs = {}
    if buf is not None:
        in_specs.append(pl.BlockSpec(memory_space=pl.MemorySpace.ANY))
        args.append(buf)
        kwargs["input_output_aliases"] = {3: 0}
        body = functools.partial(_lvl1_body_alias, k)
    return pl.pallas_call(
        body,
        grid=(nc // bc,),
        in_specs=in_specs,
        out_specs=pl.BlockSpec((rows, H),
                               lambda i: (off_blocks + i, 0)),
        out_shape=jax.ShapeDtypeStruct((_AP, H), jnp.float32),
        **kwargs,
    )(*args)


def _lvl1_body_alias(k, e2c_ref, cyc_ref, w_ref, buf_ref, out_ref):
    _lvl1_body(k, e2c_ref, cyc_ref, w_ref, out_ref)


def _eo_body(er_ref, lae_ref, eo1_ref, w2_ref, wm_ref, e11_ref, e12_ref,
             out_ref):
    eo2 = jnp.maximum(
        _dot((1.0 + e11_ref[0, 0]) * er_ref[...]
                + (1.0 + e12_ref[0, 0]) * lae_ref[...], w2_ref[...]),
        0.0,
    )
    wm = wm_ref[...]
    out_ref[...] = jnp.maximum(
        _dot(eo1_ref[...], wm[:H])
        + _dot(eo2, wm[H:]),
        0.0,
    )


def _stage_edge_out(edge_rep, lvl_aggr_e, edge_out_1, W_ec_lvl2, W_mlp,
                    eps_ec_11, eps_ec_12):
    return pl.pallas_call(
        _eo_body,
        grid=(E // _BE,),
        in_specs=[
            pl.BlockSpec((_BE, H), lambda i: (i, 0)),
            pl.BlockSpec((_BE, H), lambda i: (i, 0)),
            pl.BlockSpec((_BE, H), lambda i: (i, 0)),
            pl.BlockSpec((H, H), lambda i: (0, 0)),
            pl.BlockSpec((2 * H, H), lambda i: (0, 0)),
            pl.BlockSpec((1, 1), lambda i: (0, 0), memory_space=pltpu.SMEM),
            pl.BlockSpec((1, 1), lambda i: (0, 0), memory_space=pltpu.SMEM),
        ],
        out_specs=pl.BlockSpec((_BE, H), lambda i: (i, 0)),
        out_shape=jax.ShapeDtypeStruct((E, H), jnp.float32),
    )(edge_rep, lvl_aggr_e, edge_out_1, W_ec_lvl2, W_mlp,
      eps_ec_11.reshape(1, 1), eps_ec_12.reshape(1, 1))


def _cyc_body(k, lin_ref, e2c_ref, w_ref, eps_ref, out_ref):
    bc = e2c_ref.shape[0] // k
    e2c = e2c_ref[...]
    s = jnp.sum(e2c.reshape(bc, k, H), axis=1)
    bsum = jnp.broadcast_to(s[:, None, :], (bc, k, H)).reshape(bc * k, H)
    w = w_ref[...]
    out_ref[...] = jnp.maximum(
        (1.0 + eps_ref[0, 0])
        * _dot(lin_ref[...], w)
        + _dot(e2c, w[:H])
        + _dot(bsum, w[H:]),
        0.0,
    )


def _stage_cycle_out(k, nc, linmap, off_blocks, e2c, W_ec_lift, eps_ec_2):
    bc = 200
    rows = bc * k
    return pl.pallas_call(
        functools.partial(_cyc_body, k),
        grid=(nc // bc,),
        in_specs=[
            pl.BlockSpec((rows, 2 * H), lambda i: (off_blocks + i, 0)),
            pl.BlockSpec((rows, H), lambda i: (i, 0)),
            pl.BlockSpec((2 * H, 2 * H), lambda i: (0, 0)),
            pl.BlockSpec((1, 1), lambda i: (0, 0), memory_space=pltpu.SMEM),
        ],
        out_specs=pl.BlockSpec((rows, 2 * H), lambda i: (i, 0)),
        out_shape=jax.ShapeDtypeStruct((nc * k, 2 * H), jnp.float32),
    )(linmap, e2c, W_ec_lift, eps_ec_2.reshape(1, 1))


def kernel(node_rep, edge_rep, cycle_rep, edge_src, edge_dst, cycle5_edges,
           cycle6_edges, W_ne_lift, W_ne_lvl1, W_ne_lvl2, W_ec_lift,
           W_ec_lvl1, W_ec_lvl2, W_mlp, eps_ne_1, eps_ne_2, eps_ec_11,
           eps_ec_12, eps_ec_2):
    ne_lift = _ne_lift_sc(node_rep, edge_src, edge_dst)
    e_mid, edge_out_1 = _stage_ne(ne_lift, edge_rep, W_ne_lvl1, W_ne_lift,
                                  eps_ne_2)

    a5 = cycle5_edges.reshape(-1)
    a6 = cycle6_edges.reshape(-1)
    e2c5, e2c6 = _e2c_sc(edge_rep, a5, a6)
    cyc5 = cycle_rep[:5 * C5]
    cyc6 = cycle_rep[5 * C5:]
    lvl_c5 = _stage_lvl1(5, C5, e2c5, cyc5, W_ec_lvl1, 0)
    lvl_all = _stage_lvl1(6, C6, e2c6, cyc6, W_ec_lvl1, _P5 // 1200,
                          buf=lvl_c5)

    sentinel = jnp.int32(1 << 20)
    ape = jnp.concatenate([
        a5, jnp.full((_P5 - 5 * C5,), sentinel, jnp.int32),
        a6, jnp.full((_AP - _P5 - 6 * C6,), sentinel, jnp.int32)])
    zeros40 = jnp.zeros((40, H), jnp.float32)
    lvl_aggr_e = _edge_scatter_sc(lvl_all, ape, zeros40)
    lin2 = _linmap_sc(cycle_rep.reshape(2 * A, H), ape, zeros40)
    linmap = lin2.reshape(_LROWS, 2 * H)

    edge_out = _stage_edge_out(edge_rep, lvl_aggr_e, edge_out_1, W_ec_lvl2,
                               W_mlp, eps_ec_11, eps_ec_12)
    co5 = _stage_cycle_out(5, C5, linmap, 0, e2c5, W_ec_lift, eps_ec_2)
    co6 = _stage_cycle_out(6, C6, linmap, _P5 // 1200, e2c6, W_ec_lift,
                           eps_ec_2)
    cycle_out = jnp.concatenate([co5, co6], axis=0)

    partials = _node_aggr_sc(e_mid, edge_src, edge_dst)
    node_out = _stage_node(node_rep, partials, W_ne_lvl2, eps_ne_1)
    return (node_out, edge_out, cycle_out)


# speedup vs baseline: 1.0812x; 1.0032x over previous
import functools

import jax
import jax.numpy as jnp
from jax import lax
from jax.experimental import pallas as pl
from jax.experimental.pallas import tpu as pltpu
from jax.experimental.pallas import tpu_sc as plsc

N = 10000
E = 320000
H = 128
C5 = 10000
C6 = 10000
A = 5 * C5 + 6 * C6

_BE = 8000

_P5 = 57600
_AP = _P5 + 60032
_TA5 = _P5 // 8
_TA6 = (_AP - _P5) // 8
_LROWS = _AP + 16

_NC, _NS = 2, 16
_NW = _NC * _NS
_SC_MESH = plsc.VectorSubcoreMesh(core_axis_name="c", subcore_axis_name="s")


def _ne_lift_sc(node_rep, edge_src, edge_dst):
    epw = E // _NW
    gb = 200
    nb = epw // gb
    npr = 1000

    @functools.partial(
        pl.kernel,
        out_type=jax.ShapeDtypeStruct((E, H), jnp.float32),
        mesh=_SC_MESH,
        scratch_types=[
            pltpu.VMEM((gb,), jnp.int32),
            pltpu.VMEM((gb,), jnp.int32),
            pltpu.VMEM((gb, H), jnp.float32),
            pltpu.VMEM_SHARED((N, H), jnp.float32),
            pltpu.SemaphoreType.DMA,
            pltpu.SemaphoreType.DMA,
        ],
    )
    def k(node_hbm, src_hbm, dst_hbm, out_hbm, idx_s, idx_d, rows, nodes_sh,
          sem, sem2):
        c = lax.axis_index("c")
        s = lax.axis_index("s")
        wid = c * _NS + s

        @pl.when(s < N // npr)
        def _():
            pltpu.sync_copy(node_hbm.at[pl.ds(s * npr, npr)],
                            nodes_sh.at[pl.ds(s * npr, npr)])

        plsc.subcore_barrier()
        base = wid * epw

        def body(b, carry):
            off = base + b * gb
            d1 = pltpu.async_copy(src_hbm.at[pl.ds(off, gb)], idx_s, sem)
            d2 = pltpu.async_copy(dst_hbm.at[pl.ds(off, gb)], idx_d, sem2)
            d1.wait()
            d2.wait()
            pltpu.async_copy(nodes_sh.at[idx_s], rows, sem).wait()
            pltpu.async_copy(nodes_sh.at[idx_d], rows, sem, add=True).wait()
            pltpu.sync_copy(rows, out_hbm.at[pl.ds(off, gb)])
            return carry

        lax.fori_loop(0, nb, body, 0)

    return k(node_rep, edge_src, edge_dst)


def _node_aggr_sc(e_mid, edge_src, edge_dst):
    epw = E // _NW
    gb = 200
    nb = epw // gb
    npr = 1000
    zr = 40

    @functools.partial(
        pl.kernel,
        out_type=jax.ShapeDtypeStruct((_NC, N, H), jnp.float32),
        mesh=_SC_MESH,
        scratch_types=[
            pltpu.VMEM((gb,), jnp.int32),
            pltpu.VMEM((gb,), jnp.int32),
            pltpu.VMEM((gb, H), jnp.float32),
            pltpu.VMEM((zr, H), jnp.float32),
            pltpu.VMEM_SHARED((N, H), jnp.float32),
            pltpu.SemaphoreType.DMA,
            pltpu.SemaphoreType.DMA,
            pltpu.SemaphoreType.DMA,
        ],
    )
    def k(emid_hbm, src_hbm, dst_hbm, out_hbm, idx_s, idx_d, rows, zbuf, acc,
          sem, sem2, sem3):
        c = lax.axis_index("c")
        s = lax.axis_index("s")
        wid = c * _NS + s

        def zv(t, carry):
            i = t // 8
            j = t - i * 8
            zbuf[i, pl.ds(j * 16, 16)] = jnp.zeros((16,), jnp.float32)
            return carry

        lax.fori_loop(0, zr * 8, zv, 0)

        @pl.when(s < N // npr)
        def _():
            def zc(r, carry):
                pltpu.sync_copy(zbuf, acc.at[pl.ds(s * npr + r * zr, zr)])
                return carry

            lax.fori_loop(0, npr // zr, zc, 0)

        plsc.subcore_barrier()
        base = wid * epw

        def body(b, carry):
            off = base + b * gb
            d1 = pltpu.async_copy(src_hbm.at[pl.ds(off, gb)], idx_s, sem)
            d2 = pltpu.async_copy(dst_hbm.at[pl.ds(off, gb)], idx_d, sem2)
            d3 = pltpu.async_copy(emid_hbm.at[pl.ds(off, gb)], rows, sem3)
            d1.wait()
            d2.wait()
            d3.wait()
            d4 = pltpu.async_copy(rows, acc.at[idx_s], sem, add=True)
            d5 = pltpu.async_copy(rows, acc.at[idx_d], sem2, add=True)
            d4.wait()
            d5.wait()
            return carry

        lax.fori_loop(0, nb, body, 0)
        plsc.subcore_barrier()

        @pl.when(s < N // npr)
        def _():
            pltpu.sync_copy(acc.at[pl.ds(s * npr, npr)],
                            out_hbm.at[c, pl.ds(s * npr, npr)])

    return k(e_mid, edge_src, edge_dst)


def _e2c_sc(edge_rep, a5, a6):
    gb5, gb6 = 400, 480
    nblk5 = (5 * C5) // gb5
    nblk6 = (6 * C6) // gb6

    @functools.partial(
        pl.kernel,
        out_type=[
            jax.ShapeDtypeStruct((5 * C5, H), jnp.float32),
            jax.ShapeDtypeStruct((6 * C6, H), jnp.float32),
        ],
        mesh=_SC_MESH,
        scratch_types=[
            pltpu.VMEM((gb5,), jnp.int32),
            pltpu.VMEM((gb6,), jnp.int32),
            pltpu.VMEM((gb6, H), jnp.float32),
            pltpu.SemaphoreType.DMA,
        ],
    )
    def k(er_hbm, a5_hbm, a6_hbm, o5_hbm, o6_hbm, i5, i6, rows, sem):
        c = lax.axis_index("c")
        s = lax.axis_index("s")
        wid = c * _NS + s

        @pl.when(wid < 16)
        def _():
            def body5(t, carry):
                off = (wid + t * 16) * gb5
                pltpu.sync_copy(a5_hbm.at[pl.ds(off, gb5)], i5)
                pltpu.async_copy(er_hbm.at[i5], rows.at[pl.ds(0, gb5)],
                                 sem).wait()
                pltpu.sync_copy(rows.at[pl.ds(0, gb5)],
                                o5_hbm.at[pl.ds(off, gb5)])
                return carry

            lax.fori_loop(0, (nblk5 - wid + 15) // 16, body5, 0)

        @pl.when(wid >= 16)
        def _():
            def body6(t, carry):
                off = (wid - 16 + t * 16) * gb6
                pltpu.sync_copy(a6_hbm.at[pl.ds(off, gb6)], i6)
                pltpu.async_copy(er_hbm.at[i6], rows, sem).wait()
                pltpu.sync_copy(rows, o6_hbm.at[pl.ds(off, gb6)])
                return carry

            lax.fori_loop(0, (nblk6 - (wid - 16) + 15) // 16, body6, 0)

    return k(edge_rep, a5, a6)


def _dot(x, w):
    return jnp.dot(x, w, preferred_element_type=jnp.float32)


def _iota16():
    return lax.iota(jnp.int32, 16)


def _zero_fill(zbuf, zr):
    def zv(t, carry):
        i = t // 8
        j = t - i * 8
        zbuf[i, pl.ds(j * 16, 16)] = jnp.zeros((16,), jnp.float32)
        return carry

    lax.fori_loop(0, zr * 8, zv, 0)


def _stage_atoms(ape_hbm, aidx, s):
    @pl.when(s < 8)
    def _():
        pltpu.sync_copy(ape_hbm.at[pl.ds(s * _TA5, _TA5)],
                        aidx.at[pl.ds(0, _TA5)])

    @pl.when(s >= 8)
    def _():
        pltpu.sync_copy(ape_hbm.at[pl.ds(_P5 + (s - 8) * _TA6, _TA6)],
                        aidx.at[pl.ds(0, _TA6)])


_LANE15 = None


def _splat_last(pc):
    return jnp.take(pc, jnp.full((16,), 15, jnp.int32))


def _compress(aidx, clist, plist, la, ta, tbase, srcoff, lo, hi, it16,
              double_out=False):
    junk = la - 16

    def cbody(j, cnt_v):
        v = aidx[pl.ds(j * 16, 16)]
        m = (v >= lo) & (v < hi)
        pc = plsc.cumsum(m.astype(jnp.int32))
        dest = jnp.where(m, cnt_v + pc - 1, junk + it16)
        cval = v - lo
        pos = tbase + j * 16 + it16 - srcoff
        if double_out:
            cval = cval + cval
            pos = pos + pos
        plsc.store_scatter(clist, [dest], cval)
        plsc.store_scatter(plist, [dest], pos)
        return cnt_v + _splat_last(pc)

    return lax.fori_loop(0, ta // 16, cbody, jnp.zeros((16,), jnp.int32))


def _refresh(dst, lst, b, gb, cnt_v, padbase, shift, it16):
    def cp(jj, carry):
        g = b * gb + jj * 16
        v = lst[pl.ds(g, 16)]
        pos16 = g + it16
        dst[pl.ds(jj * 16, 16)] = jnp.where(pos16 < cnt_v, v + shift,
                                            padbase + it16)
        return carry

    lax.fori_loop(0, gb // 16, cp, 0)


_CH1 = 10240
_GB1 = 128


def _edge_scatter_sc(lvl_all, ape, zeros):
    la = _TA6 + 2 * _GB1

    @functools.partial(
        pl.kernel,
        out_type=jax.ShapeDtypeStruct((E, H), jnp.float32),
        mesh=_SC_MESH,
        compiler_params=pltpu.CompilerParams(needs_layout_passes=False),
        scratch_types=[
            pltpu.VMEM((_TA6,), jnp.int32),
            pltpu.VMEM((la,), jnp.int32),
            pltpu.VMEM((la,), jnp.int32),
            pltpu.VMEM((_GB1,), jnp.int32),
            pltpu.VMEM((_GB1,), jnp.int32),
            pltpu.VMEM((_GB1, H), jnp.float32),
            pltpu.VMEM((40, H), jnp.float32),
            pltpu.VMEM_SHARED((_CH1 + 16, H), jnp.float32),
            pltpu.SemaphoreType.DMA,
        ],
    )
    def k(lvl_hbm, ape_hbm, z_hbm, out_hbm, aidx, clist, plist, cbuf, pbuf,
          rows, zbuf, acc, sem):
        c = lax.axis_index("c")
        s = lax.axis_index("s")
        it16 = _iota16()
        pltpu.sync_copy(z_hbm, zbuf)
        _stage_atoms(ape_hbm, aidx, s)
        ta = jnp.where(s < 8, _TA5, _TA6)
        tbase = jnp.where(s < 8, s * _TA5, _P5 + (s - 8) * _TA6)

        def one_pass(p, carry):
            lo = (c * 16 + p) * _CH1
            hi = lo + _CH1
            plsc.subcore_barrier()
            zd = [
                pltpu.async_copy(
                    zbuf, acc.at[pl.ds(s * 640 + r * 40, 40)], sem)
                for r in range(16)
            ]
            cnt_v = _compress(aidx, clist, plist, la, ta, tbase, 0,
                              lo, hi, it16)
            for d in zd:
                d.wait()
            plsc.subcore_barrier()

            def sbc(b):
                return jnp.any(cnt_v > b * _GB1)

            def sb(b):
                _refresh(cbuf, clist, b, _GB1, cnt_v, _CH1, 0, it16)
                _refresh(pbuf, plist, b, _GB1, cnt_v, 0, 0, it16)
                pltpu.async_copy(lvl_hbm.at[pbuf], rows, sem).wait()
                pltpu.sync_copy(rows, acc.at[cbuf], add=True)
                return b + 1

            lax.while_loop(sbc, sb, jnp.int32(0))
            plsc.subcore_barrier()

            @pl.when(lo + s * 640 < E)
            def _():
                pltpu.sync_copy(acc.at[pl.ds(s * 640, 640)],
                                out_hbm.at[pl.ds(lo + s * 640, 640)])

            return carry

        lax.fori_loop(0, 16, one_pass, 0)

    return k(lvl_all, ape, zeros)


_CH2 = 4800
_GB2 = 128


def _linmap_sc(cyc2, ape, zeros):
    la = _TA6 + 2 * _GB2

    @functools.partial(
        pl.kernel,
        out_type=jax.ShapeDtypeStruct((2 * _LROWS, H), jnp.float32),
        mesh=_SC_MESH,
        compiler_params=pltpu.CompilerParams(needs_layout_passes=False),
        scratch_types=[
            pltpu.VMEM((_TA6,), jnp.int32),
            pltpu.VMEM((la,), jnp.int32),
            pltpu.VMEM((la,), jnp.int32),
            pltpu.VMEM((_GB2,), jnp.int32),
            pltpu.VMEM((_GB2,), jnp.int32),
            pltpu.VMEM((_GB2, H), jnp.float32),
            pltpu.VMEM((40, H), jnp.float32),
            pltpu.VMEM_SHARED((2 * _CH2 + 16, H), jnp.float32),
            pltpu.SemaphoreType.DMA,
        ],
    )
    def k(cyc_hbm, ape_hbm, z_hbm, lin_hbm, aidx, clist, plist, cbuf, pbuf,
          rows, zbuf, acc, sem):
        c = lax.axis_index("c")
        s = lax.axis_index("s")
        it16 = _iota16()
        pltpu.sync_copy(z_hbm, zbuf)
        _stage_atoms(ape_hbm, aidx, s)
        ta = jnp.where(s < 8, _TA5, _TA6)
        tbase = jnp.where(s < 8, s * _TA5, _P5 + (s - 8) * _TA6)
        srccor = jnp.where(s < 8, 0, _P5 - 5 * C5)

        def one_pass(p, carry):
            lo = (c * 34 + p) * _CH2
            hi = lo + _CH2
            plsc.subcore_barrier()
            zd = [
                pltpu.async_copy(
                    zbuf, acc.at[pl.ds(s * 600 + r * 40, 40)], sem)
                for r in range(15)
            ]
            cnt_v = _compress(aidx, clist, plist, la, ta, tbase, srccor,
                              lo, hi, it16, double_out=True)
            for d in zd:
                d.wait()
            plsc.subcore_barrier()

            def sbc(b):
                return jnp.any(cnt_v > b * _GB2)

            def sb(b):
                for half in (0, 1):
                    _refresh(cbuf, clist, b, _GB2, cnt_v, 2 * _CH2 + half,
                             half, it16)
                    _refresh(pbuf, plist, b, _GB2, cnt_v, half, half, it16)
                    pltpu.async_copy(cyc_hbm.at[pbuf], rows, sem).wait()
                    pltpu.sync_copy(rows, acc.at[cbuf], add=True)
                return b + 1

            lax.while_loop(sbc, sb, jnp.int32(0))
            plsc.subcore_barrier()

            def sb2(b):
                for half in (0, 1):
                    _refresh(cbuf, clist, b, _GB2, cnt_v, 2 * _CH2 + half,
                             half, it16)
                    _refresh(pbuf, plist, b, _GB2, cnt_v, 2 * _AP + half,
                             srccor + srccor + half, it16)
                    pltpu.sync_copy(acc.at[cbuf], rows)
                    pltpu.sync_copy(rows, lin_hbm.at[pbuf])
                return b + 1

            lax.while_loop(sbc, sb2, jnp.int32(0))
            return carry

        lax.fori_loop(0, 34 - c, one_pass, 0)

    return k(cyc2, ape, zeros)


def _ne_body(lift_ref, er_ref, w1_ref, wl_ref, eps2_ref, emid_ref, eo1_ref):
    lift = lift_ref[...]
    w1 = w1_ref[...]
    e_mid = jnp.maximum(
        _dot(lift, w1[:H])
        + _dot(er_ref[...], w1[H:]),
        0.0,
    )
    emid_ref[...] = e_mid
    eo1_ref[...] = jnp.maximum(
        _dot((1.0 + eps2_ref[0, 0]) * e_mid + lift, wl_ref[...]),
        0.0,
    )


def _stage_ne(ne_lift, edge_rep, W_ne_lvl1, W_ne_lift, eps_ne_2):
    grid = E // _BE
    return pl.pallas_call(
        _ne_body,
        grid=(grid,),
        in_specs=[
            pl.BlockSpec((_BE, H), lambda i: (i, 0)),
            pl.BlockSpec((_BE, H), lambda i: (i, 0)),
            pl.BlockSpec((2 * H, H), lambda i: (0, 0)),
            pl.BlockSpec((H, H), lambda i: (0, 0)),
            pl.BlockSpec((1, 1), lambda i: (0, 0), memory_space=pltpu.SMEM),
        ],
        out_specs=[
            pl.BlockSpec((_BE, H), lambda i: (i, 0)),
            pl.BlockSpec((_BE, H), lambda i: (i, 0)),
        ],
        out_shape=[
            jax.ShapeDtypeStruct((E, H), jnp.float32),
            jax.ShapeDtypeStruct((E, H), jnp.float32),
        ],
    )(ne_lift, edge_rep, W_ne_lvl1, W_ne_lift, eps_ne_2.reshape(1, 1))


def _node_body(nr_ref, p0_ref, p1_ref, w_ref, eps_ref, out_ref):
    x = ((1.0 + eps_ref[0, 0]) * nr_ref[...] + p0_ref[0] + p1_ref[0])
    out_ref[...] = jnp.maximum(
        _dot(x, w_ref[...]), 0.0)


def _stage_node(node_rep, partials, W_ne_lvl2, eps_ne_1):
    bn = 2000
    return pl.pallas_call(
        _node_body,
        grid=(N // bn,),
        in_specs=[
            pl.BlockSpec((bn, H), lambda i: (i, 0)),
            pl.BlockSpec((1, bn, H), lambda i: (0, i, 0)),
            pl.BlockSpec((1, bn, H), lambda i: (1, i, 0)),
            pl.BlockSpec((H, H), lambda i: (0, 0)),
            pl.BlockSpec((1, 1), lambda i: (0, 0), memory_space=pltpu.SMEM),
        ],
        out_specs=pl.BlockSpec((bn, H), lambda i: (i, 0)),
        out_shape=jax.ShapeDtypeStruct((N, H), jnp.float32),
    )(node_rep, partials, partials, W_ne_lvl2, eps_ne_1.reshape(1, 1))


def _lvl1_body(k, e2c_ref, cyc_ref, w_ref, out_ref):
    bc = e2c_ref.shape[0] // k
    e2c = e2c_ref[...]
    s = jnp.sum(e2c.reshape(bc, k, H), axis=1)
    bsum = jnp.broadcast_to(s[:, None, :], (bc, k, H)).reshape(bc * k, H)
    w = w_ref[...]
    out_ref[...] = jnp.maximum(
        _dot(e2c, w[:H])
        + _dot(bsum, w[H:2 * H])
        + _dot(cyc_ref[...], w[2 * H:]),
        0.0,
    )


def _stage_lvl1(k, nc, e2c, cyc, W_ec_lvl1, off_blocks, buf=None):
    bc = 200
    rows = bc * k
    body = functools.partial(_lvl1_body, k)
    in_specs = [
        pl.BlockSpec((rows, H), lambda i: (i, 0)),
        pl.BlockSpec((rows, 2 * H), lambda i: (i, 0)),
        pl.BlockSpec((4 * H, H), lambda i: (0, 0)),
    ]
    args = [e2c, cyc, W_ec_lvl1]
    kwargs = {}
    if buf is not None:
        in_specs.append(pl.BlockSpec(memory_space=pl.MemorySpace.ANY))
        args.append(buf)
        kwargs["input_output_aliases"] = {3: 0}
        body = functools.partial(_lvl1_body_alias, k)
    return pl.pallas_call(
        body,
        grid=(nc // bc,),
        in_specs=in_specs,
        out_specs=pl.BlockSpec((rows, H),
                               lambda i: (off_blocks + i, 0)),
        out_shape=jax.ShapeDtypeStruct((_AP, H), jnp.float32),
        **kwargs,
    )(*args)


def _lvl1_body_alias(k, e2c_ref, cyc_ref, w_ref, buf_ref, out_ref):
    _lvl1_body(k, e2c_ref, cyc_ref, w_ref, out_ref)


def _eo_body(er_ref, lae_ref, eo1_ref, w2_ref, wm_ref, e11_ref, e12_ref,
             out_ref):
    eo2 = jnp.maximum(
        _dot((1.0 + e11_ref[0, 0]) * er_ref[...]
                + (1.0 + e12_ref[0, 0]) * lae_ref[...], w2_ref[...]),
        0.0,
    )
    wm = wm_ref[...]
    out_ref[...] = jnp.maximum(
        _dot(eo1_ref[...], wm[:H])
        + _dot(eo2, wm[H:]),
        0.0,
    )


def _stage_edge_out(edge_rep, lvl_aggr_e, edge_out_1, W_ec_lvl2, W_mlp,
                    eps_ec_11, eps_ec_12):
    return pl.pallas_call(
        _eo_body,
        grid=(E // _BE,),
        in_specs=[
            pl.BlockSpec((_BE, H), lambda i: (i, 0)),
            pl.BlockSpec((_BE, H), lambda i: (i, 0)),
            pl.BlockSpec((_BE, H), lambda i: (i, 0)),
            pl.BlockSpec((H, H), lambda i: (0, 0)),
            pl.BlockSpec((2 * H, H), lambda i: (0, 0)),
            pl.BlockSpec((1, 1), lambda i: (0, 0), memory_space=pltpu.SMEM),
            pl.BlockSpec((1, 1), lambda i: (0, 0), memory_space=pltpu.SMEM),
        ],
        out_specs=pl.BlockSpec((_BE, H), lambda i: (i, 0)),
        out_shape=jax.ShapeDtypeStruct((E, H), jnp.float32),
    )(edge_rep, lvl_aggr_e, edge_out_1, W_ec_lvl2, W_mlp,
      eps_ec_11.reshape(1, 1), eps_ec_12.reshape(1, 1))


def _cyc_body(k, lin_ref, e2c_ref, w_ref, eps_ref, out_ref):
    bc = e2c_ref.shape[0] // k
    e2c = e2c_ref[...]
    s = jnp.sum(e2c.reshape(bc, k, H), axis=1)
    bsum = jnp.broadcast_to(s[:, None, :], (bc, k, H)).reshape(bc * k, H)
    w = w_ref[...]
    out_ref[...] = jnp.maximum(
        (1.0 + eps_ref[0, 0])
        * _dot(lin_ref[...], w)
        + _dot(e2c, w[:H])
        + _dot(bsum, w[H:]),
        0.0,
    )


def _stage_cycle_out(k, nc, linmap, off_blocks, e2c, W_ec_lift, eps_ec_2):
    bc = 200
    rows = bc * k
    return pl.pallas_call(
        functools.partial(_cyc_body, k),
        grid=(nc // bc,),
        in_specs=[
            pl.BlockSpec((rows, 2 * H), lambda i: (off_blocks + i, 0)),
            pl.BlockSpec((rows, H), lambda i: (i, 0)),
            pl.BlockSpec((2 * H, 2 * H), lambda i: (0, 0)),
            pl.BlockSpec((1, 1), lambda i: (0, 0), memory_space=pltpu.SMEM),
        ],
        out_specs=pl.BlockSpec((rows, 2 * H), lambda i: (i, 0)),
        out_shape=jax.ShapeDtypeStruct((nc * k, 2 * H), jnp.float32),
    )(linmap, e2c, W_ec_lift, eps_ec_2.reshape(1, 1))


def kernel(node_rep, edge_rep, cycle_rep, edge_src, edge_dst, cycle5_edges,
           cycle6_edges, W_ne_lift, W_ne_lvl1, W_ne_lvl2, W_ec_lift,
           W_ec_lvl1, W_ec_lvl2, W_mlp, eps_ne_1, eps_ne_2, eps_ec_11,
           eps_ec_12, eps_ec_2):
    ne_lift = _ne_lift_sc(node_rep, edge_src, edge_dst)
    e_mid, edge_out_1 = _stage_ne(ne_lift, edge_rep, W_ne_lvl1, W_ne_lift,
                                  eps_ne_2)

    a5 = cycle5_edges.reshape(-1)
    a6 = cycle6_edges.reshape(-1)
    e2c5, e2c6 = _e2c_sc(edge_rep, a5, a6)
    cyc5 = cycle_rep[:5 * C5]
    cyc6 = cycle_rep[5 * C5:]
    lvl_c5 = _stage_lvl1(5, C5, e2c5, cyc5, W_ec_lvl1, 0)
    lvl_all = _stage_lvl1(6, C6, e2c6, cyc6, W_ec_lvl1, _P5 // 1200,
                          buf=lvl_c5)

    sentinel = jnp.int32(1 << 20)
    ape = jnp.concatenate([
        a5, jnp.full((_P5 - 5 * C5,), sentinel, jnp.int32),
        a6, jnp.full((_AP - _P5 - 6 * C6,), sentinel, jnp.int32)])
    zeros40 = jnp.zeros((40, H), jnp.float32)
    lvl_aggr_e = _edge_scatter_sc(lvl_all, ape, zeros40)
    lin2 = _linmap_sc(cycle_rep.reshape(2 * A, H), ape, zeros40)
    linmap = lin2.reshape(_LROWS, 2 * H)

    edge_out = _stage_edge_out(edge_rep, lvl_aggr_e, edge_out_1, W_ec_lvl2,
                               W_mlp, eps_ec_11, eps_ec_12)
    co5 = _stage_cycle_out(5, C5, linmap, 0, e2c5, W_ec_lift, eps_ec_2)
    co6 = _stage_cycle_out(6, C6, linmap, _P5 // 1200, e2c6, W_ec_lift,
                           eps_ec_2)
    cycle_out = jnp.concatenate([co5, co6], axis=0)

    partials = _node_aggr_sc(e_mid, edge_src, edge_dst)
    node_out = _stage_node(node_rep, partials, W_ne_lvl2, eps_ne_1)
    return (node_out, edge_out, cycle_out)
